# Initial kernel scaffold; baseline (speedup 1.0000x reference)
import functools
import jax
import jax.numpy as jnp
from jax import lax
from jax.experimental import pallas as pl
from jax.experimental.pallas import tpu as pltpu
from jax.experimental.pallas import tpu_sc as plsc

N_NODES = 50000
D_FEAT = 128
HIDDEN = 64
LAMB = 0.8
EPS_NORM = 1e-12
EPS_COS = 1e-8

NC, NS, L = 2, 16, 16
NW = NC * NS
NACC = 50048
DUMMY = N_NODES
ZCH = 4
ZR = NACC // NS
ZB = ZR // ZCH
NCHUNK = 4
DC = 32

_mesh = plsc.VectorSubcoreMesh(core_axis_name="c", subcore_axis_name="s")
_CP = pltpu.CompilerParams(use_tc_tiling_on_sc=False,
                           needs_layout_passes=False)


@functools.partial(functools.lru_cache, maxsize=None)
def make_scatter_kernel(nb):
    @functools.partial(
        pl.kernel,
        out_type=jax.ShapeDtypeStruct((NCHUNK, NACC, DC), jnp.float32),
        mesh=_mesh,
        compiler_params=_CP,
        scratch_types=[
            pltpu.VMEM((nb, 128), jnp.int32),
            pltpu.VMEM((nb, 128), jnp.int32),
            pltpu.VMEM((128, DC), jnp.float32),
            pltpu.VMEM((ZB, DC), jnp.float32),
            pltpu.VMEM_SHARED((NACC, DC), jnp.float32),
            pltpu.SemaphoreType.DMA,
        ],
    )
    def scatter_kernel(gidx_hbm, sidx_hbm, t0, t1, t2, t3, out_hbm,
                       gidx_v, sidx_v, rows_v, zeros_v, acc_sh, sem):
        c = lax.axis_index("c")
        s = lax.axis_index("s")
        tables = (t0, t1, t2, t3)
        zv = jnp.zeros((L,), jnp.float32)

        def zb_body(i, _):
            zeros_v[i, pl.ds(0, L)] = zv
            zeros_v[i, pl.ds(L, L)] = zv
            return ()
        lax.fori_loop(0, ZB, zb_body, (), unroll=False)

        pltpu.sync_copy(gidx_hbm.at[s], gidx_v)
        pltpu.sync_copy(sidx_hbm.at[s], sidx_v)

        for ch in range(NCHUNK):
            @pl.when(c == ch // 2)
            def _():
                for z in range(ZCH):
                    pltpu.sync_copy(
                        zeros_v, acc_sh.at[pl.ds(s * ZR + z * ZB, ZB)])
                plsc.subcore_barrier()

                def body(j, _):
                    pltpu.async_copy(
                        tables[ch].at[gidx_v.at[j]], rows_v, sem).wait()
                    pltpu.sync_copy(
                        rows_v, acc_sh.at[sidx_v.at[j]], add=True)
                    return ()
                lax.fori_loop(0, nb, body, (), unroll=False)
                plsc.subcore_barrier()
                pltpu.sync_copy(acc_sh.at[pl.ds(s * ZR, ZR)],
                                out_hbm.at[ch, pl.ds(s * ZR, ZR)])
    return scatter_kernel


@functools.partial(functools.lru_cache, maxsize=None)
def make_count_kernel(nbp, nbn):
    @functools.partial(
        pl.kernel,
        out_type=[jax.ShapeDtypeStruct((NS, NACC), jnp.float32),
                  jax.ShapeDtypeStruct((NS, NACC), jnp.float32)],
        mesh=_mesh,
        compiler_params=_CP,
        scratch_types=[
            pltpu.VMEM((nbp, 128), jnp.int32),
            pltpu.VMEM((nbn, 128), jnp.int32),
            pltpu.VMEM((NACC,), jnp.float32),
        ],
    )
    def count_kernel(psidx_hbm, nsidx_hbm, outp_hbm, outn_hbm,
                     pidx_v, nidx_v, cnt_v):
        c = lax.axis_index("c")
        s = lax.axis_index("s")

        def zbody(i, _):
            cnt_v[pl.ds(i * L, L)] = jnp.zeros((L,), jnp.float32)
            return ()
        lax.fori_loop(0, NACC // L, zbody, (), unroll=False)

        ones = jnp.ones((L,), jnp.float32)

        def accumulate(idx_v, nb):
            def body(j, _):
                def inner(k, _):
                    idx = idx_v[j, pl.ds(k * L, L)]
                    plsc.addupdate_scatter(cnt_v, [idx], ones)
                    return ()
                lax.fori_loop(0, 128 // L, inner, (), unroll=False)
                return ()
            lax.fori_loop(0, nb, body, (), unroll=False)

        @pl.when(c == 0)
        def _():
            pltpu.sync_copy(psidx_hbm.at[s], pidx_v)
            accumulate(pidx_v, nbp)
            pltpu.sync_copy(cnt_v, outp_hbm.at[s])

        @pl.when(c == 1)
        def _():
            pltpu.sync_copy(nsidx_hbm.at[s], nidx_v)
            accumulate(nidx_v, nbn)
            pltpu.sync_copy(cnt_v, outn_hbm.at[s])
    return count_kernel


@functools.partial(functools.lru_cache, maxsize=None)
def make_edge_kernel(nb, is_pos):
    @functools.partial(
        pl.kernel,
        out_type=jax.ShapeDtypeStruct((NW, L), jnp.float32),
        mesh=_mesh,
        compiler_params=_CP,
        scratch_types=[
            pltpu.VMEM((nb, 128), jnp.int32),
            pltpu.VMEM((nb, 128), jnp.int32),
            pltpu.VMEM((N_NODES,), jnp.int32),
            pltpu.VMEM((128, D_FEAT), jnp.float32),
            pltpu.VMEM((128, D_FEAT), jnp.float32),
            pltpu.VMEM((L,), jnp.float32),
            pltpu.SemaphoreType.DMA,
        ],
    )
    def edge_kernel(iidx_hbm, jidx_hbm, clar_hbm, zn_hbm, out_hbm,
                    iidx_v, jidx_v, clar_v, ra_v, rb_v, acc_v, sem):
        c = lax.axis_index("c")
        s = lax.axis_index("s")
        wid = c * NS + s
        pltpu.sync_copy(iidx_hbm.at[wid], iidx_v)
        pltpu.sync_copy(jidx_hbm.at[wid], jidx_v)
        pltpu.sync_copy(clar_hbm, clar_v)

        lane = lax.iota(jnp.int32, L)
        zero = jnp.zeros((L,), jnp.float32)

        def body(j, acc):
            pltpu.async_copy(zn_hbm.at[iidx_v.at[j]], ra_v, sem).wait()
            pltpu.async_copy(zn_hbm.at[jidx_v.at[j]], rb_v, sem).wait()

            def group(g, acc):
                eidx = g * L + lane
                ci = plsc.load_gather(clar_v, [iidx_v[j, pl.ds(g * L, L)]])
                cj = plsc.load_gather(clar_v, [jidx_v[j, pl.ds(g * L, L)]])

                def dloop(d, dot):
                    dv = jnp.full((L,), d, jnp.int32)
                    va = plsc.load_gather(ra_v, [eidx, dv])
                    vb = plsc.load_gather(rb_v, [eidx, dv])
                    return dot + va * vb
                dot = lax.fori_loop(0, D_FEAT, dloop, zero, unroll=False)
                if is_pos:
                    term = jnp.where(ci != cj, jnp.maximum(dot, 0.0), 0.0)
                else:
                    term = jnp.where(ci == cj, -jnp.minimum(dot, 0.0), 0.0)
                return acc + term
            return lax.fori_loop(0, 128 // L, group, acc, unroll=False)

        acc = lax.fori_loop(0, nb, body, zero, unroll=False)
        acc_v[...] = acc
        pltpu.sync_copy(acc_v, out_hbm.at[wid])
    return edge_kernel


RB = 2500
GB = N_NODES // RB


def _l1_body(x_ref, sp_ref, sn_ref, cp_ref, cn_ref,
             wp_ref, bp_ref, wn_ref, bn_ref, hh_ref):
    x = x_ref[...]
    cp = jnp.maximum(jnp.sum(cp_ref[...], axis=0), 1.0)
    cn = jnp.maximum(jnp.sum(cn_ref[...], axis=0), 1.0)
    op = jnp.concatenate([sp_ref[i] for i in range(NCHUNK)], axis=1)
    on = jnp.concatenate([sn_ref[i] for i in range(NCHUNK)], axis=1)
    op = op / cp[:, None]
    on = on / cn[:, None]

    def head(o, w_ref, b_ref):
        w = w_ref[...]
        a = (jnp.dot(o, w[:D_FEAT], preferred_element_type=jnp.float32)
             + jnp.dot(x, w[D_FEAT:], preferred_element_type=jnp.float32)
             + b_ref[...])
        nrm = jnp.maximum(
            jnp.sqrt(jnp.sum(a * a, axis=1, keepdims=True)), EPS_NORM)
        return jnp.tanh(a / nrm)

    hp = head(op, wp_ref, bp_ref)
    hn = head(on, wn_ref, bn_ref)
    hh_ref[...] = jnp.concatenate([hp, hn], axis=1)


def layer1_tc(x, sp, sn, cp, cn, wp, bp, wn, bn):
    full = lambda shape: pl.BlockSpec(shape, lambda i: tuple(0 for _ in shape))
    return pl.pallas_call(
        _l1_body,
        grid=(GB,),
        in_specs=[
            pl.BlockSpec((RB, D_FEAT), lambda i: (i, 0)),
            pl.BlockSpec((NCHUNK, RB, DC), lambda i: (0, i, 0)),
            pl.BlockSpec((NCHUNK, RB, DC), lambda i: (0, i, 0)),
            pl.BlockSpec((NS, RB), lambda i: (0, i)),
            pl.BlockSpec((NS, RB), lambda i: (0, i)),
            full((2 * D_FEAT, HIDDEN)),
            full((1, HIDDEN)),
            full((2 * D_FEAT, HIDDEN)),
            full((1, HIDDEN)),
        ],
        out_specs=pl.BlockSpec((RB, 2 * HIDDEN), lambda i: (i, 0)),
        out_shape=jax.ShapeDtypeStruct((N_NODES, 2 * HIDDEN), jnp.float32),
    )(x, sp, sn, cp, cn, wp, bp.reshape(1, -1), wn, bn.reshape(1, -1))


def _l2_body(hh_ref, b1_ref, b2_ref, cp_ref, cn_ref,
             wp_ref, bp_ref, wn_ref, bn_ref, rw_ref, rb_ref, comm_ref,
             z_ref, zn_ref, clar_ref, rl_ref):
    hh = hh_ref[...]
    hp = hh[:, :HIDDEN]
    hn = hh[:, HIDDEN:]
    cp = jnp.maximum(jnp.sum(cp_ref[...], axis=0) + 1.0, 1.0)[:, None]
    cn = jnp.maximum(jnp.sum(cn_ref[...], axis=0) + 1.0, 1.0)[:, None]
    b1 = jnp.concatenate([b1_ref[i] for i in range(NCHUNK)], axis=1)
    b2 = jnp.concatenate([b2_ref[i] for i in range(NCHUNK)], axis=1)
    o1 = (b1 + hh) / cp
    o2 = (b2 + hh) / cn

    def head(u, w_ref, b_ref):
        a = jnp.dot(u, w_ref[...], preferred_element_type=jnp.float32) \
            + b_ref[...]
        nrm = jnp.maximum(
            jnp.sqrt(jnp.sum(a * a, axis=1, keepdims=True)), EPS_NORM)
        return jnp.tanh(a / nrm)

    hp1 = head(jnp.concatenate(
        [o1[:, :HIDDEN], o2[:, HIDDEN:], hp], axis=1), wp_ref, bp_ref)
    hn1 = head(jnp.concatenate(
        [o1[:, HIDDEN:], o2[:, :HIDDEN], hn], axis=1), wn_ref, bn_ref)
    z = jnp.concatenate([hp1, hn1], axis=1)
    z_ref[...] = z
    znrm = jnp.maximum(
        jnp.sqrt(jnp.sum(z * z, axis=1, keepdims=True)), EPS_COS)
    zn_ref[...] = z / znrm

    preds = jnp.dot(z, rw_ref[...], preferred_element_type=jnp.float32) \
        + rb_ref[...]
    colmask = lax.broadcasted_iota(jnp.int32, preds.shape, 1) < 3
    pm = jnp.where(colmask, preds, -jnp.inf)
    mx = jnp.max(pm, axis=1)
    se = jnp.sum(jnp.where(colmask, jnp.exp(pm - mx[:, None]), 0.0), axis=1)
    clar = jnp.argmax(pm, axis=1).astype(jnp.int32)
    clar_ref[...] = clar[:, None]
    pick_mask = (lax.broadcasted_iota(jnp.int32, preds.shape, 1)
                 == comm_ref[...])
    pick = jnp.sum(jnp.where(pick_mask, preds, 0.0), axis=1)
    rl_ref[0, 0] = jnp.sum(mx + jnp.log(se) - pick)


def layer2_tc(hh, b1, b2, cp, cn, wp, bp, wn, bn, rw, rbias, comm):
    full = lambda shape: pl.BlockSpec(shape, lambda i: tuple(0 for _ in shape))
    return pl.pallas_call(
        _l2_body,
        grid=(GB,),
        in_specs=[
            pl.BlockSpec((RB, 2 * HIDDEN), lambda i: (i, 0)),
            pl.BlockSpec((NCHUNK, RB, DC), lambda i: (0, i, 0)),
            pl.BlockSpec((NCHUNK, RB, DC), lambda i: (0, i, 0)),
            pl.BlockSpec((NS, RB), lambda i: (0, i)),
            pl.BlockSpec((NS, RB), lambda i: (0, i)),
            full((3 * HIDDEN, HIDDEN)),
            full((1, HIDDEN)),
            full((3 * HIDDEN, HIDDEN)),
            full((1, HIDDEN)),
            full((2 * HIDDEN, 128)),
            full((1, 128)),
            pl.BlockSpec((RB, 1), lambda i: (i, 0)),
        ],
        out_specs=[
            pl.BlockSpec((RB, 2 * HIDDEN), lambda i: (i, 0)),
            pl.BlockSpec((RB, 2 * HIDDEN), lambda i: (i, 0)),
            pl.BlockSpec((RB, 1), lambda i: (i, 0)),
            pl.BlockSpec((1, 1), lambda i: (i, 0)),
        ],
        out_shape=[
            jax.ShapeDtypeStruct((N_NODES, 2 * HIDDEN), jnp.float32),
            jax.ShapeDtypeStruct((N_NODES, 2 * HIDDEN), jnp.float32),
            jax.ShapeDtypeStruct((N_NODES, 1), jnp.int32),
            jax.ShapeDtypeStruct((GB, 1), jnp.float32),
        ],
    )(hh, b1, b2, cp, cn, wp, bp.reshape(1, -1), wn, bn.reshape(1, -1),
      rw, rbias.reshape(1, -1), comm)


def finalize_tc(rl, pparts, nparts, e_pos, e_neg):
    def body(rl_ref, pp_ref, np_ref, out_ref):
        reg = jnp.sum(rl_ref[...]) / N_NODES
        sim1 = jnp.sum(pp_ref[...]) / e_pos
        sim2 = jnp.sum(np_ref[...]) / e_neg
        out_ref[0, 0] = LAMB * reg + (1.0 - LAMB) * (sim1 + sim2)
    return pl.pallas_call(
        body,
        out_shape=jax.ShapeDtypeStruct((1, 1), jnp.float32),
    )(rl, pparts, nparts)


def _pad_to(a, n, fill):
    return jnp.concatenate(
        [a, jnp.full((n - a.shape[0],), fill, a.dtype)])


def _chunk_table(t):
    tc = jnp.moveaxis(t.reshape(N_NODES, NCHUNK, DC), 1, 0)
    return [tc[i] for i in range(NCHUNK)]


def kernel(positive_edges, negative_edges, comm, X,
           W_pos_base, b_pos_base, W_neg_base, b_neg_base,
           W_pos_deep, b_pos_deep, W_neg_deep, b_neg_deep,
           regression_weights, regression_bias):
    e_pos = positive_edges.shape[1]
    e_neg = negative_edges.shape[1]

    def prep_scatter(edges):
        e = edges.shape[1]
        epad = -(-e // (NS * 128)) * (NS * 128)
        gidx = _pad_to(edges[1], epad, 0)
        sidx = jnp.where(edges[0] == edges[1], DUMMY, edges[0])
        sidx = _pad_to(sidx, epad, DUMMY)
        nb = epad // (NS * 128)
        return (gidx.reshape(NS, nb, 128), sidx.reshape(NS, nb, 128), nb)

    pg, ps, nbp = prep_scatter(positive_edges)
    ng, nsx, nbn = prep_scatter(negative_edges)

    cntp, cntn = make_count_kernel(nbp, nbn)(ps, nsx)

    xch = _chunk_table(X)
    sp = make_scatter_kernel(nbp)(pg, ps, *xch)
    sn = make_scatter_kernel(nbn)(ng, nsx, *xch)

    hh = layer1_tc(X, sp, sn, cntp, cntn,
                   W_pos_base, b_pos_base, W_neg_base, b_neg_base)

    hch = _chunk_table(hh)
    b1 = make_scatter_kernel(nbp)(pg, ps, *hch)
    b2 = make_scatter_kernel(nbn)(ng, nsx, *hch)

    rwpad = jnp.zeros((2 * HIDDEN, 128), jnp.float32).at[:, :3].set(
        regression_weights)
    rbpad = jnp.zeros((128,), jnp.float32).at[:3].set(regression_bias)
    z, zn, clar2d, rl = layer2_tc(
        hh, b1, b2, cntp, cntn,
        W_pos_deep, b_pos_deep, W_neg_deep, b_neg_deep,
        rwpad, rbpad, comm.astype(jnp.int32).reshape(-1, 1))

    def prep_edge(edges):
        e = edges.shape[1]
        epad = -(-e // (NW * 128)) * (NW * 128)
        i = _pad_to(edges[0], epad, 0)
        j = _pad_to(edges[1], epad, 0)
        nb = epad // (NW * 128)
        return (i.reshape(NW, nb, 128), j.reshape(NW, nb, 128), nb)

    pi, pj, nbp2 = prep_edge(positive_edges)
    ni, nj, nbn2 = prep_edge(negative_edges)
    clar1d = clar2d[:, 0]
    pparts = make_edge_kernel(nbp2, True)(pi, pj, clar1d, zn)
    nparts = make_edge_kernel(nbn2, False)(ni, nj, clar1d, zn)

    loss = finalize_tc(rl, pparts, nparts, e_pos, e_neg)
    return loss[0, 0], z, clar1d


# speedup vs baseline: 4.0046x; 4.0046x over previous
import functools
import jax
import jax.numpy as jnp
from jax import lax
from jax.experimental import pallas as pl
from jax.experimental.pallas import tpu as pltpu
from jax.experimental.pallas import tpu_sc as plsc

N_NODES = 50000
D_FEAT = 128
HIDDEN = 64
LAMB = 0.8
EPS_NORM = 1e-12
EPS_COS = 1e-8

NC, NS, L = 2, 16, 16
NW = NC * NS
NACC = 50048
DUMMY = N_NODES
ZCH = 8
ZR = NACC // NS
ZB = ZR // ZCH
NCHUNK = 4
DC = 32
SB = 8

@functools.cache
def _get_mesh():
    return plsc.VectorSubcoreMesh(core_axis_name="c", subcore_axis_name="s")
_CP = pltpu.CompilerParams(use_tc_tiling_on_sc=False,
                           needs_layout_passes=False)


@functools.lru_cache(maxsize=None)
def make_scatter_kernel(nsb):
    @functools.partial(
        pl.kernel,
        out_type=jax.ShapeDtypeStruct((NCHUNK, NACC, DC), jnp.float32),
        mesh=_get_mesh(),
        compiler_params=_CP,
        scratch_types=[
            pltpu.VMEM((SB, 128), jnp.int32),
            pltpu.VMEM((SB, 128), jnp.int32),
            pltpu.VMEM((128, DC), jnp.float32),
            pltpu.VMEM((ZB, DC), jnp.float32),
            pltpu.VMEM_SHARED((NACC, DC), jnp.float32),
            pltpu.SemaphoreType.DMA,
        ],
    )
    def scatter_kernel(gidx_hbm, sidx_hbm, t0, t1, t2, t3, out_hbm,
                       gidx_v, sidx_v, rows_v, zeros_v, acc_sh, sem):
        c = lax.axis_index("c")
        s = lax.axis_index("s")
        tables = (t0, t1, t2, t3)
        zv = jnp.zeros((L,), jnp.float32)

        def zb_body(i, _):
            zeros_v[i, pl.ds(0, L)] = zv
            zeros_v[i, pl.ds(L, L)] = zv
            return ()
        lax.fori_loop(0, ZB, zb_body, (), unroll=False)

        for ch in range(NCHUNK):
            @pl.when(c == ch // 2)
            def _():
                for z in range(ZCH):
                    pltpu.sync_copy(
                        zeros_v, acc_sh.at[pl.ds(s * ZR + z * ZB, ZB)])
                plsc.subcore_barrier()

                def sb_body(b, _):
                    pltpu.sync_copy(
                        gidx_hbm.at[s, pl.ds(b * SB, SB)], gidx_v)
                    pltpu.sync_copy(
                        sidx_hbm.at[s, pl.ds(b * SB, SB)], sidx_v)

                    def body(j, _):
                        pltpu.async_copy(
                            tables[ch].at[gidx_v.at[j]], rows_v, sem).wait()
                        pltpu.sync_copy(
                            rows_v, acc_sh.at[sidx_v.at[j]], add=True)
                        return ()
                    lax.fori_loop(0, SB, body, (), unroll=False)
                    return ()
                lax.fori_loop(0, nsb, sb_body, (), unroll=False)
                plsc.subcore_barrier()
                pltpu.sync_copy(acc_sh.at[pl.ds(s * ZR, ZR)],
                                out_hbm.at[ch, pl.ds(s * ZR, ZR)])
    return scatter_kernel


@functools.lru_cache(maxsize=None)
def make_count_kernel(nbp, nbn):
    @functools.partial(
        pl.kernel,
        out_type=[jax.ShapeDtypeStruct((NS, NACC), jnp.float32),
                  jax.ShapeDtypeStruct((NS, NACC), jnp.float32)],
        mesh=_get_mesh(),
        compiler_params=_CP,
        scratch_types=[
            pltpu.VMEM((nbp, 128), jnp.int32),
            pltpu.VMEM((nbn, 128), jnp.int32),
            pltpu.VMEM((NACC,), jnp.float32),
        ],
    )
    def count_kernel(psidx_hbm, nsidx_hbm, outp_hbm, outn_hbm,
                     pidx_v, nidx_v, cnt_v):
        c = lax.axis_index("c")
        s = lax.axis_index("s")

        def zbody(i, _):
            cnt_v[pl.ds(i * L, L)] = jnp.zeros((L,), jnp.float32)
            return ()
        lax.fori_loop(0, NACC // L, zbody, (), unroll=False)

        ones = jnp.ones((L,), jnp.float32)

        def accumulate(idx_v, nb):
            def body(j, _):
                def inner(k, _):
                    idx = idx_v[j, pl.ds(k * L, L)]
                    plsc.addupdate_scatter(cnt_v, [idx], ones)
                    return ()
                lax.fori_loop(0, 128 // L, inner, (), unroll=False)
                return ()
            lax.fori_loop(0, nb, body, (), unroll=False)

        @pl.when(c == 0)
        def _():
            pltpu.sync_copy(psidx_hbm.at[s], pidx_v)
            accumulate(pidx_v, nbp)
            pltpu.sync_copy(cnt_v, outp_hbm.at[s])

        @pl.when(c == 1)
        def _():
            pltpu.sync_copy(nsidx_hbm.at[s], nidx_v)
            accumulate(nidx_v, nbn)
            pltpu.sync_copy(cnt_v, outn_hbm.at[s])
    return count_kernel


@functools.lru_cache(maxsize=None)
def make_edge_kernel(nb, is_pos):
    @functools.partial(
        pl.kernel,
        out_type=jax.ShapeDtypeStruct((NW, L), jnp.float32),
        mesh=_get_mesh(),
        compiler_params=_CP,
        scratch_types=[
            pltpu.VMEM((nb, 128), jnp.int32),
            pltpu.VMEM((nb, 128), jnp.int32),
            pltpu.VMEM((N_NODES,), jnp.int32),
            pltpu.VMEM((128, D_FEAT), jnp.float32),
            pltpu.VMEM((128, D_FEAT), jnp.float32),
            pltpu.VMEM((L,), jnp.float32),
            pltpu.SemaphoreType.DMA,
        ],
    )
    def edge_kernel(iidx_hbm, jidx_hbm, clar_hbm, zn_hbm, out_hbm,
                    iidx_v, jidx_v, clar_v, ra_v, rb_v, acc_v, sem):
        c = lax.axis_index("c")
        s = lax.axis_index("s")
        wid = c * NS + s
        pltpu.sync_copy(iidx_hbm.at[wid], iidx_v)
        pltpu.sync_copy(jidx_hbm.at[wid], jidx_v)
        pltpu.sync_copy(clar_hbm, clar_v)

        lane = lax.iota(jnp.int32, L)
        zero = jnp.zeros((L,), jnp.float32)

        def body(j, acc):
            pltpu.async_copy(zn_hbm.at[iidx_v.at[j]], ra_v, sem).wait()
            pltpu.async_copy(zn_hbm.at[jidx_v.at[j]], rb_v, sem).wait()

            def group(g, acc):
                eidx = g * L + lane
                ci = plsc.load_gather(clar_v, [iidx_v[j, pl.ds(g * L, L)]])
                cj = plsc.load_gather(clar_v, [jidx_v[j, pl.ds(g * L, L)]])

                def dloop(d, dot):
                    dv = jnp.full((L,), d, jnp.int32)
                    va = plsc.load_gather(ra_v, [eidx, dv])
                    vb = plsc.load_gather(rb_v, [eidx, dv])
                    return dot + va * vb
                dot = lax.fori_loop(0, D_FEAT, dloop, zero, unroll=False)
                if is_pos:
                    term = jnp.where(ci != cj, jnp.maximum(dot, 0.0), 0.0)
                else:
                    term = jnp.where(ci == cj, -jnp.minimum(dot, 0.0), 0.0)
                return acc + term
            return lax.fori_loop(0, 128 // L, group, acc, unroll=False)

        acc = lax.fori_loop(0, nb, body, zero, unroll=False)
        acc_v[...] = acc
        pltpu.sync_copy(acc_v, out_hbm.at[wid])
    return edge_kernel


NPAD = 50048
RB = 2176
GB = NPAD // RB


def _l1_body(x_ref, sp_ref, sn_ref, cp_ref, cn_ref,
             wp_ref, bp_ref, wn_ref, bn_ref, hh_ref):
    x = x_ref[...]
    cp = jnp.maximum(jnp.sum(cp_ref[...], axis=0), 1.0)
    cn = jnp.maximum(jnp.sum(cn_ref[...], axis=0), 1.0)
    op = jnp.concatenate([sp_ref[i] for i in range(NCHUNK)], axis=1)
    on = jnp.concatenate([sn_ref[i] for i in range(NCHUNK)], axis=1)
    op = op / cp[:, None]
    on = on / cn[:, None]

    def head(o, w_ref, b_ref):
        w = w_ref[...]
        a = (jnp.dot(o, w[:D_FEAT], preferred_element_type=jnp.float32)
             + jnp.dot(x, w[D_FEAT:], preferred_element_type=jnp.float32)
             + b_ref[...])
        nrm = jnp.maximum(
            jnp.sqrt(jnp.sum(a * a, axis=1, keepdims=True)), EPS_NORM)
        return jnp.tanh(a / nrm)

    hp = head(op, wp_ref, bp_ref)
    hn = head(on, wn_ref, bn_ref)
    hh_ref[...] = jnp.concatenate([hp, hn], axis=1)


def layer1_tc(x, sp, sn, cp, cn, wp, bp, wn, bn):
    full = lambda shape: pl.BlockSpec(shape, lambda i: tuple(0 for _ in shape))
    return pl.pallas_call(
        _l1_body,
        grid=(GB,),
        in_specs=[
            pl.BlockSpec((RB, D_FEAT), lambda i: (i, 0)),
            pl.BlockSpec((NCHUNK, RB, DC), lambda i: (0, i, 0)),
            pl.BlockSpec((NCHUNK, RB, DC), lambda i: (0, i, 0)),
            pl.BlockSpec((NS, RB), lambda i: (0, i)),
            pl.BlockSpec((NS, RB), lambda i: (0, i)),
            full((2 * D_FEAT, HIDDEN)),
            full((1, HIDDEN)),
            full((2 * D_FEAT, HIDDEN)),
            full((1, HIDDEN)),
        ],
        out_specs=pl.BlockSpec((RB, 2 * HIDDEN), lambda i: (i, 0)),
        out_shape=jax.ShapeDtypeStruct((NPAD, 2 * HIDDEN), jnp.float32),
    )(x, sp, sn, cp, cn, wp, bp.reshape(1, -1), wn, bn.reshape(1, -1))


def _l2_body(hh_ref, b1_ref, b2_ref, cp_ref, cn_ref,
             wp_ref, bp_ref, wn_ref, bn_ref, rw_ref, rb_ref, comm_ref,
             z_ref, zn_ref, clar_ref, rl_ref):
    hh = hh_ref[...]
    hp = hh[:, :HIDDEN]
    hn = hh[:, HIDDEN:]
    cp = jnp.maximum(jnp.sum(cp_ref[...], axis=0) + 1.0, 1.0)[:, None]
    cn = jnp.maximum(jnp.sum(cn_ref[...], axis=0) + 1.0, 1.0)[:, None]
    b1 = jnp.concatenate([b1_ref[i] for i in range(NCHUNK)], axis=1)
    b2 = jnp.concatenate([b2_ref[i] for i in range(NCHUNK)], axis=1)
    o1 = (b1 + hh) / cp
    o2 = (b2 + hh) / cn

    def head(u, w_ref, b_ref):
        a = jnp.dot(u, w_ref[...], preferred_element_type=jnp.float32) \
            + b_ref[...]
        nrm = jnp.maximum(
            jnp.sqrt(jnp.sum(a * a, axis=1, keepdims=True)), EPS_NORM)
        return jnp.tanh(a / nrm)

    hp1 = head(jnp.concatenate(
        [o1[:, :HIDDEN], o2[:, HIDDEN:], hp], axis=1), wp_ref, bp_ref)
    hn1 = head(jnp.concatenate(
        [o1[:, HIDDEN:], o2[:, :HIDDEN], hn], axis=1), wn_ref, bn_ref)
    z = jnp.concatenate([hp1, hn1], axis=1)
    z_ref[...] = z
    znrm = jnp.maximum(
        jnp.sqrt(jnp.sum(z * z, axis=1, keepdims=True)), EPS_COS)
    zn_ref[...] = z / znrm

    preds = jnp.dot(z, rw_ref[...], preferred_element_type=jnp.float32) \
        + rb_ref[...]
    colmask = lax.broadcasted_iota(jnp.int32, preds.shape, 1) < 3
    pm = jnp.where(colmask, preds, -jnp.inf)
    mx = jnp.max(pm, axis=1)
    se = jnp.sum(jnp.where(colmask, jnp.exp(pm - mx[:, None]), 0.0), axis=1)
    clar = jnp.argmax(pm, axis=1).astype(jnp.int32)
    clar_ref[...] = clar[:, None]
    pick_mask = (lax.broadcasted_iota(jnp.int32, preds.shape, 1)
                 == comm_ref[...])
    pick = jnp.sum(jnp.where(pick_mask, preds, 0.0), axis=1)
    row = (pl.program_id(0) * RB
           + lax.broadcasted_iota(jnp.int32, (RB, 1), 0))
    val = jnp.sum(jnp.where(
        row < N_NODES, (mx + jnp.log(se) - pick)[:, None], 0.0))

    @pl.when(pl.program_id(0) == 0)
    def _():
        rl_ref[...] = jnp.zeros((1, 1), jnp.float32)
    rl_ref[...] += jnp.full((1, 1), val, jnp.float32)


def layer2_tc(hh, b1, b2, cp, cn, wp, bp, wn, bn, rw, rbias, comm):
    full = lambda shape: pl.BlockSpec(shape, lambda i: tuple(0 for _ in shape))
    return pl.pallas_call(
        _l2_body,
        grid=(GB,),
        in_specs=[
            pl.BlockSpec((RB, 2 * HIDDEN), lambda i: (i, 0)),
            pl.BlockSpec((NCHUNK, RB, DC), lambda i: (0, i, 0)),
            pl.BlockSpec((NCHUNK, RB, DC), lambda i: (0, i, 0)),
            pl.BlockSpec((NS, RB), lambda i: (0, i)),
            pl.BlockSpec((NS, RB), lambda i: (0, i)),
            full((3 * HIDDEN, HIDDEN)),
            full((1, HIDDEN)),
            full((3 * HIDDEN, HIDDEN)),
            full((1, HIDDEN)),
            full((2 * HIDDEN, 128)),
            full((1, 128)),
            pl.BlockSpec((RB, 1), lambda i: (i, 0)),
        ],
        out_specs=[
            pl.BlockSpec((RB, 2 * HIDDEN), lambda i: (i, 0)),
            pl.BlockSpec((RB, 2 * HIDDEN), lambda i: (i, 0)),
            pl.BlockSpec((RB, 1), lambda i: (i, 0)),
            pl.BlockSpec((1, 1), lambda i: (0, 0)),
        ],
        out_shape=[
            jax.ShapeDtypeStruct((NPAD, 2 * HIDDEN), jnp.float32),
            jax.ShapeDtypeStruct((NPAD, 2 * HIDDEN), jnp.float32),
            jax.ShapeDtypeStruct((NPAD, 1), jnp.int32),
            jax.ShapeDtypeStruct((1, 1), jnp.float32),
        ],
    )(hh, b1, b2, cp, cn, wp, bp.reshape(1, -1), wn, bn.reshape(1, -1),
      rw, rbias.reshape(1, -1), comm)


def finalize_tc(rl, pparts, nparts, e_pos, e_neg):
    def body(rl_ref, pp_ref, np_ref, out_ref):
        reg = jnp.sum(rl_ref[...]) / N_NODES
        sim1 = jnp.sum(pp_ref[...]) / e_pos
        sim2 = jnp.sum(np_ref[...]) / e_neg
        out_ref[...] = jnp.full(
            (1, 1), LAMB * reg + (1.0 - LAMB) * (sim1 + sim2), jnp.float32)
    return pl.pallas_call(
        body,
        out_shape=jax.ShapeDtypeStruct((1, 1), jnp.float32),
    )(rl, pparts, nparts)


def _pad_to(a, n, fill):
    return jnp.concatenate(
        [a, jnp.full((n - a.shape[0],), fill, a.dtype)])


def _chunk_table(t):
    tc = jnp.moveaxis(t.reshape(t.shape[0], NCHUNK, DC), 1, 0)
    return [tc[i] for i in range(NCHUNK)]


def kernel(positive_edges, negative_edges, comm, X,
           W_pos_base, b_pos_base, W_neg_base, b_neg_base,
           W_pos_deep, b_pos_deep, W_neg_deep, b_neg_deep,
           regression_weights, regression_bias):
    e_pos = positive_edges.shape[1]
    e_neg = negative_edges.shape[1]

    def prep_scatter(edges):
        e = edges.shape[1]
        gran = NS * 128 * SB
        epad = -(-e // gran) * gran
        gidx = _pad_to(edges[1], epad, 0)
        sidx = jnp.where(edges[0] == edges[1], DUMMY, edges[0])
        sidx = _pad_to(sidx, epad, DUMMY)
        nb = epad // (NS * 128)
        return (gidx.reshape(NS, nb, 128), sidx.reshape(NS, nb, 128),
                nb // SB)

    pg, ps, nsbp = prep_scatter(positive_edges)
    ng, nsx, nsbn = prep_scatter(negative_edges)

    cntp, cntn = make_count_kernel(nsbp * SB, nsbn * SB)(ps, nsx)

    xch = _chunk_table(X)
    sp = make_scatter_kernel(nsbp)(pg, ps, *xch)
    sn = make_scatter_kernel(nsbn)(ng, nsx, *xch)

    xp = jnp.concatenate(
        [X, jnp.zeros((NPAD - N_NODES, D_FEAT), jnp.float32)])
    hh = layer1_tc(xp, sp, sn, cntp, cntn,
                   W_pos_base, b_pos_base, W_neg_base, b_neg_base)

    hch = _chunk_table(hh)
    b1 = make_scatter_kernel(nsbp)(pg, ps, *hch)
    b2 = make_scatter_kernel(nsbn)(ng, nsx, *hch)

    rwpad = jnp.zeros((2 * HIDDEN, 128), jnp.float32).at[:, :3].set(
        regression_weights)
    rbpad = jnp.zeros((128,), jnp.float32).at[:3].set(regression_bias)
    commp = _pad_to(comm.astype(jnp.int32), NPAD, 0).reshape(-1, 1)
    z, zn, clar2d, rl = layer2_tc(
        hh, b1, b2, cntp, cntn,
        W_pos_deep, b_pos_deep, W_neg_deep, b_neg_deep,
        rwpad, rbpad, commp)

    def prep_edge(edges):
        e = edges.shape[1]
        epad = -(-e // (NW * 128)) * (NW * 128)
        i = _pad_to(edges[0], epad, 0)
        j = _pad_to(edges[1], epad, 0)
        nb = epad // (NW * 128)
        return (i.reshape(NW, nb, 128), j.reshape(NW, nb, 128), nb)

    pi, pj, nbp2 = prep_edge(positive_edges)
    ni, nj, nbn2 = prep_edge(negative_edges)
    clar1d = clar2d[:N_NODES, 0]
    pparts = make_edge_kernel(nbp2, True)(pi, pj, clar1d, zn)
    nparts = make_edge_kernel(nbn2, False)(ni, nj, clar1d, zn)

    loss = finalize_tc(rl, pparts, nparts, e_pos, e_neg)
    return loss[0, 0], z[:N_NODES], clar1d


# speedup vs baseline: 4.4933x; 1.1220x over previous
import functools
import jax
import jax.numpy as jnp
from jax import lax
from jax.experimental import pallas as pl
from jax.experimental.pallas import tpu as pltpu
from jax.experimental.pallas import tpu_sc as plsc

N_NODES = 50000
D_FEAT = 128
HIDDEN = 64
LAMB = 0.8
EPS_NORM = 1e-12
EPS_COS = 1e-8

NC, NS, L = 2, 16, 16
NW = NC * NS
NACC = 50048
DUMMY = N_NODES
ZCH = 8
ZR = NACC // NS
ZB = ZR // ZCH
NCHUNK = 4
DC = 32
SB = 8

@functools.cache
def _get_mesh():
    return plsc.VectorSubcoreMesh(core_axis_name="c", subcore_axis_name="s")
_CP = pltpu.CompilerParams(use_tc_tiling_on_sc=False,
                           needs_layout_passes=False)


@functools.lru_cache(maxsize=None)
def make_scatter_kernel(nsb):
    @functools.partial(
        pl.kernel,
        out_type=jax.ShapeDtypeStruct((NCHUNK, NACC, DC), jnp.float32),
        mesh=_get_mesh(),
        compiler_params=_CP,
        scratch_types=[
            pltpu.VMEM((SB, 128), jnp.int32),
            pltpu.VMEM((SB, 128), jnp.int32),
            pltpu.VMEM((128, DC), jnp.float32),
            pltpu.VMEM((ZB, DC), jnp.float32),
            pltpu.VMEM_SHARED((NACC, DC), jnp.float32),
            pltpu.SemaphoreType.DMA,
        ],
    )
    def scatter_kernel(gidx_hbm, sidx_hbm, t0, t1, t2, t3, out_hbm,
                       gidx_v, sidx_v, rows_v, zeros_v, acc_sh, sem):
        c = lax.axis_index("c")
        s = lax.axis_index("s")
        tables = (t0, t1, t2, t3)
        zv = jnp.zeros((L,), jnp.float32)

        def zb_body(i, _):
            zeros_v[i, pl.ds(0, L)] = zv
            zeros_v[i, pl.ds(L, L)] = zv
            return ()
        lax.fori_loop(0, ZB, zb_body, (), unroll=False)

        for ch in range(NCHUNK):
            @pl.when(c == ch // 2)
            def _():
                for z in range(ZCH):
                    pltpu.sync_copy(
                        zeros_v, acc_sh.at[pl.ds(s * ZR + z * ZB, ZB)])
                plsc.subcore_barrier()

                def sb_body(b, _):
                    pltpu.sync_copy(
                        gidx_hbm.at[s, pl.ds(b * SB, SB)], gidx_v)
                    pltpu.sync_copy(
                        sidx_hbm.at[s, pl.ds(b * SB, SB)], sidx_v)

                    def body(j, _):
                        pltpu.async_copy(
                            tables[ch].at[gidx_v.at[j]], rows_v, sem).wait()
                        pltpu.sync_copy(
                            rows_v, acc_sh.at[sidx_v.at[j]], add=True)
                        return ()
                    lax.fori_loop(0, SB, body, (), unroll=False)
                    return ()
                lax.fori_loop(0, nsb, sb_body, (), unroll=False)
                plsc.subcore_barrier()
                pltpu.sync_copy(acc_sh.at[pl.ds(s * ZR, ZR)],
                                out_hbm.at[ch, pl.ds(s * ZR, ZR)])
    return scatter_kernel


@functools.lru_cache(maxsize=None)
def make_count_kernel(nbp, nbn):
    @functools.partial(
        pl.kernel,
        out_type=[jax.ShapeDtypeStruct((NS, NACC), jnp.float32),
                  jax.ShapeDtypeStruct((NS, NACC), jnp.float32)],
        mesh=_get_mesh(),
        compiler_params=_CP,
        scratch_types=[
            pltpu.VMEM((nbp, 128), jnp.int32),
            pltpu.VMEM((nbn, 128), jnp.int32),
            pltpu.VMEM((NACC,), jnp.float32),
        ],
    )
    def count_kernel(psidx_hbm, nsidx_hbm, outp_hbm, outn_hbm,
                     pidx_v, nidx_v, cnt_v):
        c = lax.axis_index("c")
        s = lax.axis_index("s")

        def zbody(i, _):
            cnt_v[pl.ds(i * L, L)] = jnp.zeros((L,), jnp.float32)
            return ()
        lax.fori_loop(0, NACC // L, zbody, (), unroll=False)

        ones = jnp.ones((L,), jnp.float32)

        def accumulate(idx_v, nb):
            def body(j, _):
                def inner(k, _):
                    idx = idx_v[j, pl.ds(k * L, L)]
                    plsc.addupdate_scatter(cnt_v, [idx], ones)
                    return ()
                lax.fori_loop(0, 128 // L, inner, (), unroll=False)
                return ()
            lax.fori_loop(0, nb, body, (), unroll=False)

        @pl.when(c == 0)
        def _():
            pltpu.sync_copy(psidx_hbm.at[s], pidx_v)
            accumulate(pidx_v, nbp)
            pltpu.sync_copy(cnt_v, outp_hbm.at[s])

        @pl.when(c == 1)
        def _():
            pltpu.sync_copy(nsidx_hbm.at[s], nidx_v)
            accumulate(nidx_v, nbn)
            pltpu.sync_copy(cnt_v, outn_hbm.at[s])
    return count_kernel


@functools.lru_cache(maxsize=None)
def make_edge_kernel(nb, is_pos):
    @functools.partial(
        pl.kernel,
        out_type=jax.ShapeDtypeStruct((NW, L), jnp.float32),
        mesh=_get_mesh(),
        compiler_params=_CP,
        scratch_types=[
            pltpu.VMEM((nb, 128), jnp.int32),
            pltpu.VMEM((nb, 128), jnp.int32),
            pltpu.VMEM((2, 128, D_FEAT), jnp.float32),
            pltpu.VMEM((2, 128, D_FEAT), jnp.float32),
            pltpu.VMEM((2, 128, 8), jnp.int32),
            pltpu.VMEM((2, 128, 8), jnp.int32),
            pltpu.VMEM((L,), jnp.float32),
            pltpu.SemaphoreType.DMA,
            pltpu.SemaphoreType.DMA,
        ],
    )
    def edge_kernel(iidx_hbm, jidx_hbm, clar_hbm, zn_hbm, out_hbm,
                    iidx_v, jidx_v, ra_v, rb_v, ci_v, cj_v, acc_v,
                    sem0, sem1):
        c = lax.axis_index("c")
        s = lax.axis_index("s")
        wid = c * NS + s
        pltpu.sync_copy(iidx_hbm.at[wid], iidx_v)
        pltpu.sync_copy(jidx_hbm.at[wid], jidx_v)

        lane = lax.iota(jnp.int32, L)
        zero = jnp.zeros((L,), jnp.float32)
        zlane = jnp.zeros((L,), jnp.int32)
        sems = (sem0, sem1)

        def issue(g, b, sem):
            pltpu.async_copy(zn_hbm.at[iidx_v.at[g]], ra_v.at[b], sem)
            pltpu.async_copy(zn_hbm.at[jidx_v.at[g]], rb_v.at[b], sem)
            pltpu.async_copy(clar_hbm.at[iidx_v.at[g]], ci_v.at[b], sem)
            pltpu.async_copy(clar_hbm.at[jidx_v.at[g]], cj_v.at[b], sem)

        def drain(g, b, sem):
            pltpu.make_async_copy(
                zn_hbm.at[iidx_v.at[g]], ra_v.at[b], sem).wait()
            pltpu.make_async_copy(
                zn_hbm.at[jidx_v.at[g]], rb_v.at[b], sem).wait()
            pltpu.make_async_copy(
                clar_hbm.at[iidx_v.at[g]], ci_v.at[b], sem).wait()
            pltpu.make_async_copy(
                clar_hbm.at[jidx_v.at[g]], cj_v.at[b], sem).wait()

        issue(0, 0, sem0)

        def outer(go, acc):
            for b in range(2):
                g = 2 * go + b

                @pl.when(g + 1 < nb)
                def _():
                    issue(g + 1, 1 - b, sems[1 - b])
                drain(g, b, sems[b])

                def group(gg, acc):
                    eidx = gg * L + lane
                    ci = plsc.load_gather(ci_v.at[b], [eidx, zlane])
                    cj = plsc.load_gather(cj_v.at[b], [eidx, zlane])

                    def dloop(d, dot):
                        dv = jnp.full((L,), d, jnp.int32)
                        va = plsc.load_gather(ra_v.at[b], [eidx, dv])
                        vb = plsc.load_gather(rb_v.at[b], [eidx, dv])
                        return dot + va * vb
                    dot = lax.fori_loop(0, D_FEAT, dloop, zero, unroll=8)
                    if is_pos:
                        term = jnp.where(ci != cj, jnp.maximum(dot, 0.0), 0.0)
                    else:
                        term = jnp.where(ci == cj, -jnp.minimum(dot, 0.0), 0.0)
                    return acc + term
                acc = lax.fori_loop(0, 128 // L, group, acc, unroll=False)
            return acc

        acc = lax.fori_loop(0, nb // 2, outer, zero, unroll=False)
        acc_v[...] = acc
        pltpu.sync_copy(acc_v, out_hbm.at[wid])
    return edge_kernel


NPAD = 50048
RB = 2176
GB = NPAD // RB


def _l1_body(x_ref, sp_ref, sn_ref, cp_ref, cn_ref,
             wp_ref, bp_ref, wn_ref, bn_ref, hh_ref):
    x = x_ref[...]
    cp = jnp.maximum(jnp.sum(cp_ref[...], axis=0), 1.0)
    cn = jnp.maximum(jnp.sum(cn_ref[...], axis=0), 1.0)
    op = jnp.concatenate([sp_ref[i] for i in range(NCHUNK)], axis=1)
    on = jnp.concatenate([sn_ref[i] for i in range(NCHUNK)], axis=1)
    op = op / cp[:, None]
    on = on / cn[:, None]

    def head(o, w_ref, b_ref):
        w = w_ref[...]
        a = (jnp.dot(o, w[:D_FEAT], preferred_element_type=jnp.float32)
             + jnp.dot(x, w[D_FEAT:], preferred_element_type=jnp.float32)
             + b_ref[...])
        nrm = jnp.maximum(
            jnp.sqrt(jnp.sum(a * a, axis=1, keepdims=True)), EPS_NORM)
        return jnp.tanh(a / nrm)

    hp = head(op, wp_ref, bp_ref)
    hn = head(on, wn_ref, bn_ref)
    hh_ref[...] = jnp.concatenate([hp, hn], axis=1)


def layer1_tc(x, sp, sn, cp, cn, wp, bp, wn, bn):
    full = lambda shape: pl.BlockSpec(shape, lambda i: tuple(0 for _ in shape))
    return pl.pallas_call(
        _l1_body,
        grid=(GB,),
        in_specs=[
            pl.BlockSpec((RB, D_FEAT), lambda i: (i, 0)),
            pl.BlockSpec((NCHUNK, RB, DC), lambda i: (0, i, 0)),
            pl.BlockSpec((NCHUNK, RB, DC), lambda i: (0, i, 0)),
            pl.BlockSpec((NS, RB), lambda i: (0, i)),
            pl.BlockSpec((NS, RB), lambda i: (0, i)),
            full((2 * D_FEAT, HIDDEN)),
            full((1, HIDDEN)),
            full((2 * D_FEAT, HIDDEN)),
            full((1, HIDDEN)),
        ],
        out_specs=pl.BlockSpec((RB, 2 * HIDDEN), lambda i: (i, 0)),
        out_shape=jax.ShapeDtypeStruct((NPAD, 2 * HIDDEN), jnp.float32),
    )(x, sp, sn, cp, cn, wp, bp.reshape(1, -1), wn, bn.reshape(1, -1))


def _l2_body(hh_ref, b1_ref, b2_ref, cp_ref, cn_ref,
             wp_ref, bp_ref, wn_ref, bn_ref, rw_ref, rb_ref, comm_ref,
             z_ref, zn_ref, clar_ref, rl_ref):
    hh = hh_ref[...]
    hp = hh[:, :HIDDEN]
    hn = hh[:, HIDDEN:]
    cp = jnp.maximum(jnp.sum(cp_ref[...], axis=0) + 1.0, 1.0)[:, None]
    cn = jnp.maximum(jnp.sum(cn_ref[...], axis=0) + 1.0, 1.0)[:, None]
    b1 = jnp.concatenate([b1_ref[i] for i in range(NCHUNK)], axis=1)
    b2 = jnp.concatenate([b2_ref[i] for i in range(NCHUNK)], axis=1)
    o1 = (b1 + hh) / cp
    o2 = (b2 + hh) / cn

    def head(u, w_ref, b_ref):
        a = jnp.dot(u, w_ref[...], preferred_element_type=jnp.float32) \
            + b_ref[...]
        nrm = jnp.maximum(
            jnp.sqrt(jnp.sum(a * a, axis=1, keepdims=True)), EPS_NORM)
        return jnp.tanh(a / nrm)

    hp1 = head(jnp.concatenate(
        [o1[:, :HIDDEN], o2[:, HIDDEN:], hp], axis=1), wp_ref, bp_ref)
    hn1 = head(jnp.concatenate(
        [o1[:, HIDDEN:], o2[:, :HIDDEN], hn], axis=1), wn_ref, bn_ref)
    z = jnp.concatenate([hp1, hn1], axis=1)
    z_ref[...] = z
    znrm = jnp.maximum(
        jnp.sqrt(jnp.sum(z * z, axis=1, keepdims=True)), EPS_COS)
    zn_ref[...] = z / znrm

    preds = jnp.dot(z, rw_ref[...], preferred_element_type=jnp.float32) \
        + rb_ref[...]
    colmask = lax.broadcasted_iota(jnp.int32, preds.shape, 1) < 3
    pm = jnp.where(colmask, preds, -jnp.inf)
    mx = jnp.max(pm, axis=1)
    se = jnp.sum(jnp.where(colmask, jnp.exp(pm - mx[:, None]), 0.0), axis=1)
    clar = jnp.argmax(pm, axis=1).astype(jnp.int32)
    clar_ref[...] = jnp.broadcast_to(clar[:, None], (RB, 8))
    pick_mask = (lax.broadcasted_iota(jnp.int32, preds.shape, 1)
                 == comm_ref[...])
    pick = jnp.sum(jnp.where(pick_mask, preds, 0.0), axis=1)
    row = (pl.program_id(0) * RB
           + lax.broadcasted_iota(jnp.int32, (RB, 1), 0))
    val = jnp.sum(jnp.where(
        row < N_NODES, (mx + jnp.log(se) - pick)[:, None], 0.0))

    @pl.when(pl.program_id(0) == 0)
    def _():
        rl_ref[...] = jnp.zeros((1, 1), jnp.float32)
    rl_ref[...] += jnp.full((1, 1), val, jnp.float32)


def layer2_tc(hh, b1, b2, cp, cn, wp, bp, wn, bn, rw, rbias, comm):
    full = lambda shape: pl.BlockSpec(shape, lambda i: tuple(0 for _ in shape))
    return pl.pallas_call(
        _l2_body,
        grid=(GB,),
        in_specs=[
            pl.BlockSpec((RB, 2 * HIDDEN), lambda i: (i, 0)),
            pl.BlockSpec((NCHUNK, RB, DC), lambda i: (0, i, 0)),
            pl.BlockSpec((NCHUNK, RB, DC), lambda i: (0, i, 0)),
            pl.BlockSpec((NS, RB), lambda i: (0, i)),
            pl.BlockSpec((NS, RB), lambda i: (0, i)),
            full((3 * HIDDEN, HIDDEN)),
            full((1, HIDDEN)),
            full((3 * HIDDEN, HIDDEN)),
            full((1, HIDDEN)),
            full((2 * HIDDEN, 128)),
            full((1, 128)),
            pl.BlockSpec((RB, 1), lambda i: (i, 0)),
        ],
        out_specs=[
            pl.BlockSpec((RB, 2 * HIDDEN), lambda i: (i, 0)),
            pl.BlockSpec((RB, 2 * HIDDEN), lambda i: (i, 0)),
            pl.BlockSpec((RB, 8), lambda i: (i, 0)),
            pl.BlockSpec((1, 1), lambda i: (0, 0)),
        ],
        out_shape=[
            jax.ShapeDtypeStruct((NPAD, 2 * HIDDEN), jnp.float32),
            jax.ShapeDtypeStruct((NPAD, 2 * HIDDEN), jnp.float32),
            jax.ShapeDtypeStruct((NPAD, 8), jnp.int32),
            jax.ShapeDtypeStruct((1, 1), jnp.float32),
        ],
    )(hh, b1, b2, cp, cn, wp, bp.reshape(1, -1), wn, bn.reshape(1, -1),
      rw, rbias.reshape(1, -1), comm)


def finalize_tc(rl, pparts, nparts, e_pos, e_neg):
    def body(rl_ref, pp_ref, np_ref, out_ref):
        reg = jnp.sum(rl_ref[...]) / N_NODES
        sim1 = jnp.sum(pp_ref[...]) / e_pos
        sim2 = jnp.sum(np_ref[...]) / e_neg
        out_ref[...] = jnp.full(
            (1, 1), LAMB * reg + (1.0 - LAMB) * (sim1 + sim2), jnp.float32)
    return pl.pallas_call(
        body,
        out_shape=jax.ShapeDtypeStruct((1, 1), jnp.float32),
    )(rl, pparts, nparts)


def _pad_to(a, n, fill):
    return jnp.concatenate(
        [a, jnp.full((n - a.shape[0],), fill, a.dtype)])


def _chunk_table(t):
    tc = jnp.moveaxis(t.reshape(t.shape[0], NCHUNK, DC), 1, 0)
    return [tc[i] for i in range(NCHUNK)]


def kernel(positive_edges, negative_edges, comm, X,
           W_pos_base, b_pos_base, W_neg_base, b_neg_base,
           W_pos_deep, b_pos_deep, W_neg_deep, b_neg_deep,
           regression_weights, regression_bias):
    e_pos = positive_edges.shape[1]
    e_neg = negative_edges.shape[1]

    def prep_scatter(edges):
        e = edges.shape[1]
        gran = NS * 128 * SB
        epad = -(-e // gran) * gran
        gidx = _pad_to(edges[1], epad, 0)
        sidx = jnp.where(edges[0] == edges[1], DUMMY, edges[0])
        sidx = _pad_to(sidx, epad, DUMMY)
        nb = epad // (NS * 128)
        return (gidx.reshape(NS, nb, 128), sidx.reshape(NS, nb, 128),
                nb // SB)

    pg, ps, nsbp = prep_scatter(positive_edges)
    ng, nsx, nsbn = prep_scatter(negative_edges)

    cntp, cntn = make_count_kernel(nsbp * SB, nsbn * SB)(ps, nsx)

    xch = _chunk_table(X)
    sp = make_scatter_kernel(nsbp)(pg, ps, *xch)
    sn = make_scatter_kernel(nsbn)(ng, nsx, *xch)

    xp = jnp.concatenate(
        [X, jnp.zeros((NPAD - N_NODES, D_FEAT), jnp.float32)])
    hh = layer1_tc(xp, sp, sn, cntp, cntn,
                   W_pos_base, b_pos_base, W_neg_base, b_neg_base)

    hch = _chunk_table(hh)
    b1 = make_scatter_kernel(nsbp)(pg, ps, *hch)
    b2 = make_scatter_kernel(nsbn)(ng, nsx, *hch)

    rwpad = jnp.zeros((2 * HIDDEN, 128), jnp.float32).at[:, :3].set(
        regression_weights)
    rbpad = jnp.zeros((128,), jnp.float32).at[:3].set(regression_bias)
    commp = _pad_to(comm.astype(jnp.int32), NPAD, 0).reshape(-1, 1)
    z, zn, clar2d, rl = layer2_tc(
        hh, b1, b2, cntp, cntn,
        W_pos_deep, b_pos_deep, W_neg_deep, b_neg_deep,
        rwpad, rbpad, commp)

    def prep_edge(edges):
        e = edges.shape[1]
        gran = NW * 128 * 2
        epad = -(-e // gran) * gran
        i = _pad_to(edges[0], epad, 0)
        j = _pad_to(edges[1], epad, 0)
        nb = epad // (NW * 128)
        return (i.reshape(NW, nb, 128), j.reshape(NW, nb, 128), nb)

    pi, pj, nbp2 = prep_edge(positive_edges)
    ni, nj, nbn2 = prep_edge(negative_edges)
    pparts = make_edge_kernel(nbp2, True)(pi, pj, clar2d, zn)
    nparts = make_edge_kernel(nbn2, False)(ni, nj, clar2d, zn)

    clar1d = clar2d[:N_NODES, 0]

    loss = finalize_tc(rl, pparts, nparts, e_pos, e_neg)
    return loss[0, 0], z[:N_NODES], clar1d


# speedup vs baseline: 4.9563x; 1.1031x over previous
import functools
import jax
import jax.numpy as jnp
from jax import lax
from jax.experimental import pallas as pl
from jax.experimental.pallas import tpu as pltpu
from jax.experimental.pallas import tpu_sc as plsc

N_NODES = 50000
D_FEAT = 128
HIDDEN = 64
LAMB = 0.8
EPS_NORM = 1e-12
EPS_COS = 1e-8

NC, NS, L = 2, 16, 16
NW = NC * NS
NACC = 50048
DUMMY = N_NODES
ZCH = 23
ZR = NACC // NS
ZB = ZR // ZCH
NCHUNK = 4
DC = 32
SB = 8

@functools.cache
def _get_mesh():
    return plsc.VectorSubcoreMesh(core_axis_name="c", subcore_axis_name="s")
_CP = pltpu.CompilerParams(use_tc_tiling_on_sc=False,
                           needs_layout_passes=False)


@functools.lru_cache(maxsize=None)
def make_scatter_kernel(nsb):
    @functools.partial(
        pl.kernel,
        out_type=jax.ShapeDtypeStruct((NCHUNK, NACC, DC), jnp.float32),
        mesh=_get_mesh(),
        compiler_params=_CP,
        scratch_types=[
            pltpu.VMEM((SB, 128), jnp.int32),
            pltpu.VMEM((SB, 128), jnp.int32),
            pltpu.VMEM((2, 128, DC), jnp.float32),
            pltpu.VMEM((ZB, DC), jnp.float32),
            pltpu.VMEM_SHARED((NACC, DC), jnp.float32),
            pltpu.SemaphoreType.DMA,
            pltpu.SemaphoreType.DMA,
        ],
    )
    def scatter_kernel(gidx_hbm, sidx_hbm, t0, t1, t2, t3, out_hbm,
                       gidx_v, sidx_v, rows_v, zeros_v, acc_sh, sem0, sem1):
        c = lax.axis_index("c")
        s = lax.axis_index("s")
        tables = (t0, t1, t2, t3)
        sems = (sem0, sem1)
        zv = jnp.zeros((L,), jnp.float32)

        def zb_body(i, _):
            zeros_v[i, pl.ds(0, L)] = zv
            zeros_v[i, pl.ds(L, L)] = zv
            return ()
        lax.fori_loop(0, ZB, zb_body, (), unroll=False)

        for ch in range(NCHUNK):
            @pl.when(c == ch // 2)
            def _():
                for z in range(ZCH):
                    pltpu.sync_copy(
                        zeros_v, acc_sh.at[pl.ds(s * ZR + z * ZB, ZB)])
                plsc.subcore_barrier()
                tab = tables[ch]

                def sb_body(b, _):
                    pltpu.sync_copy(
                        gidx_hbm.at[s, pl.ds(b * SB, SB)], gidx_v)
                    pltpu.sync_copy(
                        sidx_hbm.at[s, pl.ds(b * SB, SB)], sidx_v)

                    pltpu.async_copy(
                        tab.at[gidx_v.at[0]], rows_v.at[0], sem0)

                    def body(jo, _):
                        for p in range(2):
                            j = 2 * jo + p

                            @pl.when(j + 1 < SB)
                            def _():
                                pltpu.async_copy(
                                    tab.at[gidx_v.at[j + 1]],
                                    rows_v.at[1 - p], sems[1 - p])
                            pltpu.make_async_copy(
                                tab.at[gidx_v.at[j]], rows_v.at[p],
                                sems[p]).wait()
                            pltpu.sync_copy(
                                rows_v.at[p], acc_sh.at[sidx_v.at[j]],
                                add=True)
                        return ()
                    lax.fori_loop(0, SB // 2, body, (), unroll=False)
                    return ()
                lax.fori_loop(0, nsb, sb_body, (), unroll=False)
                plsc.subcore_barrier()
                pltpu.sync_copy(acc_sh.at[pl.ds(s * ZR, ZR)],
                                out_hbm.at[ch, pl.ds(s * ZR, ZR)])
    return scatter_kernel


@functools.lru_cache(maxsize=None)
def make_count_kernel(nbp, nbn):
    @functools.partial(
        pl.kernel,
        out_type=[jax.ShapeDtypeStruct((NS, NACC), jnp.float32),
                  jax.ShapeDtypeStruct((NS, NACC), jnp.float32)],
        mesh=_get_mesh(),
        compiler_params=_CP,
        scratch_types=[
            pltpu.VMEM((nbp, 128), jnp.int32),
            pltpu.VMEM((nbn, 128), jnp.int32),
            pltpu.VMEM((NACC,), jnp.float32),
        ],
    )
    def count_kernel(psidx_hbm, nsidx_hbm, outp_hbm, outn_hbm,
                     pidx_v, nidx_v, cnt_v):
        c = lax.axis_index("c")
        s = lax.axis_index("s")

        def zbody(i, _):
            cnt_v[pl.ds(i * L, L)] = jnp.zeros((L,), jnp.float32)
            return ()
        lax.fori_loop(0, NACC // L, zbody, (), unroll=False)

        ones = jnp.ones((L,), jnp.float32)

        def accumulate(idx_v, nb):
            def body(j, _):
                def inner(k, _):
                    idx = idx_v[j, pl.ds(k * L, L)]
                    plsc.addupdate_scatter(cnt_v, [idx], ones)
                    return ()
                lax.fori_loop(0, 128 // L, inner, (), unroll=False)
                return ()
            lax.fori_loop(0, nb, body, (), unroll=False)

        @pl.when(c == 0)
        def _():
            pltpu.sync_copy(psidx_hbm.at[s], pidx_v)
            accumulate(pidx_v, nbp)
            pltpu.sync_copy(cnt_v, outp_hbm.at[s])

        @pl.when(c == 1)
        def _():
            pltpu.sync_copy(nsidx_hbm.at[s], nidx_v)
            accumulate(nidx_v, nbn)
            pltpu.sync_copy(cnt_v, outn_hbm.at[s])
    return count_kernel


@functools.lru_cache(maxsize=None)
def make_edge_kernel(nb, is_pos):
    @functools.partial(
        pl.kernel,
        out_type=jax.ShapeDtypeStruct((NW, L), jnp.float32),
        mesh=_get_mesh(),
        compiler_params=_CP,
        scratch_types=[
            pltpu.VMEM((nb, 128), jnp.int32),
            pltpu.VMEM((nb, 128), jnp.int32),
            pltpu.VMEM((2, 128, D_FEAT), jnp.float32),
            pltpu.VMEM((2, 128, D_FEAT), jnp.float32),
            pltpu.VMEM((2, 128, 8), jnp.int32),
            pltpu.VMEM((2, 128, 8), jnp.int32),
            pltpu.VMEM((L,), jnp.float32),
            pltpu.SemaphoreType.DMA,
            pltpu.SemaphoreType.DMA,
        ],
    )
    def edge_kernel(iidx_hbm, jidx_hbm, clar_hbm, zn_hbm, out_hbm,
                    iidx_v, jidx_v, ra_v, rb_v, ci_v, cj_v, acc_v,
                    sem0, sem1):
        c = lax.axis_index("c")
        s = lax.axis_index("s")
        wid = c * NS + s
        pltpu.sync_copy(iidx_hbm.at[wid], iidx_v)
        pltpu.sync_copy(jidx_hbm.at[wid], jidx_v)

        lane = lax.iota(jnp.int32, L)
        zero = jnp.zeros((L,), jnp.float32)
        zlane = jnp.zeros((L,), jnp.int32)
        sems = (sem0, sem1)

        def issue(g, b, sem):
            pltpu.async_copy(zn_hbm.at[iidx_v.at[g]], ra_v.at[b], sem)
            pltpu.async_copy(zn_hbm.at[jidx_v.at[g]], rb_v.at[b], sem)
            pltpu.async_copy(clar_hbm.at[iidx_v.at[g]], ci_v.at[b], sem)
            pltpu.async_copy(clar_hbm.at[jidx_v.at[g]], cj_v.at[b], sem)

        def drain(g, b, sem):
            pltpu.make_async_copy(
                zn_hbm.at[iidx_v.at[g]], ra_v.at[b], sem).wait()
            pltpu.make_async_copy(
                zn_hbm.at[jidx_v.at[g]], rb_v.at[b], sem).wait()
            pltpu.make_async_copy(
                clar_hbm.at[iidx_v.at[g]], ci_v.at[b], sem).wait()
            pltpu.make_async_copy(
                clar_hbm.at[jidx_v.at[g]], cj_v.at[b], sem).wait()

        issue(0, 0, sem0)

        def outer(go, acc):
            for b in range(2):
                g = 2 * go + b

                @pl.when(g + 1 < nb)
                def _():
                    issue(g + 1, 1 - b, sems[1 - b])
                drain(g, b, sems[b])

                def group(gg, acc):
                    eidx = gg * L + lane
                    ci = plsc.load_gather(ci_v.at[b], [eidx, zlane])
                    cj = plsc.load_gather(cj_v.at[b], [eidx, zlane])

                    def dloop(d, dot):
                        dv = jnp.full((L,), d, jnp.int32)
                        va = plsc.load_gather(ra_v.at[b], [eidx, dv])
                        vb = plsc.load_gather(rb_v.at[b], [eidx, dv])
                        return dot + va * vb
                    dot = lax.fori_loop(0, D_FEAT, dloop, zero, unroll=8)
                    if is_pos:
                        term = jnp.where(ci != cj, jnp.maximum(dot, 0.0), 0.0)
                    else:
                        term = jnp.where(ci == cj, -jnp.minimum(dot, 0.0), 0.0)
                    return acc + term
                acc = lax.fori_loop(0, 128 // L, group, acc, unroll=False)
            return acc

        acc = lax.fori_loop(0, nb // 2, outer, zero, unroll=False)
        acc_v[...] = acc
        pltpu.sync_copy(acc_v, out_hbm.at[wid])
    return edge_kernel


NPAD = 50048
RB = 2176
GB = NPAD // RB


def _l1_body(x_ref, sp_ref, sn_ref, cp_ref, cn_ref,
             wp_ref, bp_ref, wn_ref, bn_ref, hh_ref):
    x = x_ref[...]
    cp = jnp.maximum(jnp.sum(cp_ref[...], axis=0), 1.0)
    cn = jnp.maximum(jnp.sum(cn_ref[...], axis=0), 1.0)
    op = jnp.concatenate([sp_ref[i] for i in range(NCHUNK)], axis=1)
    on = jnp.concatenate([sn_ref[i] for i in range(NCHUNK)], axis=1)
    op = op / cp[:, None]
    on = on / cn[:, None]

    def head(o, w_ref, b_ref):
        w = w_ref[...]
        a = (jnp.dot(o, w[:D_FEAT], preferred_element_type=jnp.float32)
             + jnp.dot(x, w[D_FEAT:], preferred_element_type=jnp.float32)
             + b_ref[...])
        nrm = jnp.maximum(
            jnp.sqrt(jnp.sum(a * a, axis=1, keepdims=True)), EPS_NORM)
        return jnp.tanh(a / nrm)

    hp = head(op, wp_ref, bp_ref)
    hn = head(on, wn_ref, bn_ref)
    hh_ref[...] = jnp.concatenate([hp, hn], axis=1)


def layer1_tc(x, sp, sn, cp, cn, wp, bp, wn, bn):
    full = lambda shape: pl.BlockSpec(shape, lambda i: tuple(0 for _ in shape))
    return pl.pallas_call(
        _l1_body,
        grid=(GB,),
        in_specs=[
            pl.BlockSpec((RB, D_FEAT), lambda i: (i, 0)),
            pl.BlockSpec((NCHUNK, RB, DC), lambda i: (0, i, 0)),
            pl.BlockSpec((NCHUNK, RB, DC), lambda i: (0, i, 0)),
            pl.BlockSpec((NS, RB), lambda i: (0, i)),
            pl.BlockSpec((NS, RB), lambda i: (0, i)),
            full((2 * D_FEAT, HIDDEN)),
            full((1, HIDDEN)),
            full((2 * D_FEAT, HIDDEN)),
            full((1, HIDDEN)),
        ],
        out_specs=pl.BlockSpec((RB, 2 * HIDDEN), lambda i: (i, 0)),
        out_shape=jax.ShapeDtypeStruct((NPAD, 2 * HIDDEN), jnp.float32),
    )(x, sp, sn, cp, cn, wp, bp.reshape(1, -1), wn, bn.reshape(1, -1))


def _l2_body(hh_ref, b1_ref, b2_ref, cp_ref, cn_ref,
             wp_ref, bp_ref, wn_ref, bn_ref, rw_ref, rb_ref, comm_ref,
             z_ref, zn_ref, clar_ref, rl_ref):
    hh = hh_ref[...]
    hp = hh[:, :HIDDEN]
    hn = hh[:, HIDDEN:]
    cp = jnp.maximum(jnp.sum(cp_ref[...], axis=0) + 1.0, 1.0)[:, None]
    cn = jnp.maximum(jnp.sum(cn_ref[...], axis=0) + 1.0, 1.0)[:, None]
    b1 = jnp.concatenate([b1_ref[i] for i in range(NCHUNK)], axis=1)
    b2 = jnp.concatenate([b2_ref[i] for i in range(NCHUNK)], axis=1)
    o1 = (b1 + hh) / cp
    o2 = (b2 + hh) / cn

    def head(u, w_ref, b_ref):
        a = jnp.dot(u, w_ref[...], preferred_element_type=jnp.float32) \
            + b_ref[...]
        nrm = jnp.maximum(
            jnp.sqrt(jnp.sum(a * a, axis=1, keepdims=True)), EPS_NORM)
        return jnp.tanh(a / nrm)

    hp1 = head(jnp.concatenate(
        [o1[:, :HIDDEN], o2[:, HIDDEN:], hp], axis=1), wp_ref, bp_ref)
    hn1 = head(jnp.concatenate(
        [o1[:, HIDDEN:], o2[:, :HIDDEN], hn], axis=1), wn_ref, bn_ref)
    z = jnp.concatenate([hp1, hn1], axis=1)
    z_ref[...] = z
    znrm = jnp.maximum(
        jnp.sqrt(jnp.sum(z * z, axis=1, keepdims=True)), EPS_COS)
    zn_ref[...] = z / znrm

    preds = jnp.dot(z, rw_ref[...], preferred_element_type=jnp.float32) \
        + rb_ref[...]
    colmask = lax.broadcasted_iota(jnp.int32, preds.shape, 1) < 3
    pm = jnp.where(colmask, preds, -jnp.inf)
    mx = jnp.max(pm, axis=1)
    se = jnp.sum(jnp.where(colmask, jnp.exp(pm - mx[:, None]), 0.0), axis=1)
    clar = jnp.argmax(pm, axis=1).astype(jnp.int32)
    clar_ref[...] = jnp.broadcast_to(clar[:, None], (RB, 8))
    pick_mask = (lax.broadcasted_iota(jnp.int32, preds.shape, 1)
                 == comm_ref[...])
    pick = jnp.sum(jnp.where(pick_mask, preds, 0.0), axis=1)
    row = (pl.program_id(0) * RB
           + lax.broadcasted_iota(jnp.int32, (RB, 1), 0))
    val = jnp.sum(jnp.where(
        row < N_NODES, (mx + jnp.log(se) - pick)[:, None], 0.0))

    @pl.when(pl.program_id(0) == 0)
    def _():
        rl_ref[...] = jnp.zeros((1, 1), jnp.float32)
    rl_ref[...] += jnp.full((1, 1), val, jnp.float32)


def layer2_tc(hh, b1, b2, cp, cn, wp, bp, wn, bn, rw, rbias, comm):
    full = lambda shape: pl.BlockSpec(shape, lambda i: tuple(0 for _ in shape))
    return pl.pallas_call(
        _l2_body,
        grid=(GB,),
        in_specs=[
            pl.BlockSpec((RB, 2 * HIDDEN), lambda i: (i, 0)),
            pl.BlockSpec((NCHUNK, RB, DC), lambda i: (0, i, 0)),
            pl.BlockSpec((NCHUNK, RB, DC), lambda i: (0, i, 0)),
            pl.BlockSpec((NS, RB), lambda i: (0, i)),
            pl.BlockSpec((NS, RB), lambda i: (0, i)),
            full((3 * HIDDEN, HIDDEN)),
            full((1, HIDDEN)),
            full((3 * HIDDEN, HIDDEN)),
            full((1, HIDDEN)),
            full((2 * HIDDEN, 128)),
            full((1, 128)),
            pl.BlockSpec((RB, 1), lambda i: (i, 0)),
        ],
        out_specs=[
            pl.BlockSpec((RB, 2 * HIDDEN), lambda i: (i, 0)),
            pl.BlockSpec((RB, 2 * HIDDEN), lambda i: (i, 0)),
            pl.BlockSpec((RB, 8), lambda i: (i, 0)),
            pl.BlockSpec((1, 1), lambda i: (0, 0)),
        ],
        out_shape=[
            jax.ShapeDtypeStruct((NPAD, 2 * HIDDEN), jnp.float32),
            jax.ShapeDtypeStruct((NPAD, 2 * HIDDEN), jnp.float32),
            jax.ShapeDtypeStruct((NPAD, 8), jnp.int32),
            jax.ShapeDtypeStruct((1, 1), jnp.float32),
        ],
    )(hh, b1, b2, cp, cn, wp, bp.reshape(1, -1), wn, bn.reshape(1, -1),
      rw, rbias.reshape(1, -1), comm)


def finalize_tc(rl, pparts, nparts, e_pos, e_neg):
    def body(rl_ref, pp_ref, np_ref, out_ref):
        reg = jnp.sum(rl_ref[...]) / N_NODES
        sim1 = jnp.sum(pp_ref[...]) / e_pos
        sim2 = jnp.sum(np_ref[...]) / e_neg
        out_ref[...] = jnp.full(
            (1, 1), LAMB * reg + (1.0 - LAMB) * (sim1 + sim2), jnp.float32)
    return pl.pallas_call(
        body,
        out_shape=jax.ShapeDtypeStruct((1, 1), jnp.float32),
    )(rl, pparts, nparts)


def _pad_to(a, n, fill):
    return jnp.concatenate(
        [a, jnp.full((n - a.shape[0],), fill, a.dtype)])


def _chunk_table(t):
    tc = jnp.moveaxis(t.reshape(t.shape[0], NCHUNK, DC), 1, 0)
    return [tc[i] for i in range(NCHUNK)]


def kernel(positive_edges, negative_edges, comm, X,
           W_pos_base, b_pos_base, W_neg_base, b_neg_base,
           W_pos_deep, b_pos_deep, W_neg_deep, b_neg_deep,
           regression_weights, regression_bias):
    e_pos = positive_edges.shape[1]
    e_neg = negative_edges.shape[1]

    def prep_scatter(edges):
        e = edges.shape[1]
        gran = NS * 128 * SB
        epad = -(-e // gran) * gran
        gidx = _pad_to(edges[1], epad, 0)
        sidx = jnp.where(edges[0] == edges[1], DUMMY, edges[0])
        sidx = _pad_to(sidx, epad, DUMMY)
        nb = epad // (NS * 128)
        return (gidx.reshape(NS, nb, 128), sidx.reshape(NS, nb, 128),
                nb // SB)

    pg, ps, nsbp = prep_scatter(positive_edges)
    ng, nsx, nsbn = prep_scatter(negative_edges)

    cntp, cntn = make_count_kernel(nsbp * SB, nsbn * SB)(ps, nsx)

    xch = _chunk_table(X)
    sp = make_scatter_kernel(nsbp)(pg, ps, *xch)
    sn = make_scatter_kernel(nsbn)(ng, nsx, *xch)

    xp = jnp.concatenate(
        [X, jnp.zeros((NPAD - N_NODES, D_FEAT), jnp.float32)])
    hh = layer1_tc(xp, sp, sn, cntp, cntn,
                   W_pos_base, b_pos_base, W_neg_base, b_neg_base)

    hch = _chunk_table(hh)
    b1 = make_scatter_kernel(nsbp)(pg, ps, *hch)
    b2 = make_scatter_kernel(nsbn)(ng, nsx, *hch)

    rwpad = jnp.zeros((2 * HIDDEN, 128), jnp.float32).at[:, :3].set(
        regression_weights)
    rbpad = jnp.zeros((128,), jnp.float32).at[:3].set(regression_bias)
    commp = _pad_to(comm.astype(jnp.int32), NPAD, 0).reshape(-1, 1)
    z, zn, clar2d, rl = layer2_tc(
        hh, b1, b2, cntp, cntn,
        W_pos_deep, b_pos_deep, W_neg_deep, b_neg_deep,
        rwpad, rbpad, commp)

    def prep_edge(edges):
        e = edges.shape[1]
        gran = NW * 128 * 2
        epad = -(-e // gran) * gran
        i = _pad_to(edges[0], epad, 0)
        j = _pad_to(edges[1], epad, 0)
        nb = epad // (NW * 128)
        return (i.reshape(NW, nb, 128), j.reshape(NW, nb, 128), nb)

    pi, pj, nbp2 = prep_edge(positive_edges)
    ni, nj, nbn2 = prep_edge(negative_edges)
    pparts = make_edge_kernel(nbp2, True)(pi, pj, clar2d, zn)
    nparts = make_edge_kernel(nbn2, False)(ni, nj, clar2d, zn)

    clar1d = clar2d[:N_NODES, 0]

    loss = finalize_tc(rl, pparts, nparts, e_pos, e_neg)
    return loss[0, 0], z[:N_NODES], clar1d


# speedup vs baseline: 5.0773x; 1.0244x over previous
import functools
import jax
import jax.numpy as jnp
from jax import lax
from jax.experimental import pallas as pl
from jax.experimental.pallas import tpu as pltpu
from jax.experimental.pallas import tpu_sc as plsc

N_NODES = 50000
D_FEAT = 128
HIDDEN = 64
LAMB = 0.8
EPS_NORM = 1e-12
EPS_COS = 1e-8

NC, NS, L = 2, 16, 16
NW = NC * NS
NACC = 50048
DUMMY = N_NODES
ZCH = 46
ZR = NACC // NS
ZB = ZR // ZCH
NBUF = 4
NCHUNK = 4
DC = 32
SB = 8

@functools.cache
def _get_mesh():
    return plsc.VectorSubcoreMesh(core_axis_name="c", subcore_axis_name="s")
_CP = pltpu.CompilerParams(use_tc_tiling_on_sc=False,
                           needs_layout_passes=False)


@functools.lru_cache(maxsize=None)
def make_scatter_kernel(nsb):
    @functools.partial(
        pl.kernel,
        out_type=jax.ShapeDtypeStruct((NCHUNK, NACC, DC), jnp.float32),
        mesh=_get_mesh(),
        compiler_params=_CP,
        scratch_types=[
            pltpu.VMEM((SB, 128), jnp.int32),
            pltpu.VMEM((SB, 128), jnp.int32),
            pltpu.VMEM((NBUF, 128, DC), jnp.float32),
            pltpu.VMEM((ZB, DC), jnp.float32),
            pltpu.VMEM_SHARED((NACC, DC), jnp.float32),
            pltpu.SemaphoreType.DMA,
            pltpu.SemaphoreType.DMA,
            pltpu.SemaphoreType.DMA,
            pltpu.SemaphoreType.DMA,
        ],
    )
    def scatter_kernel(gidx_hbm, sidx_hbm, t0, t1, t2, t3, out_hbm,
                       gidx_v, sidx_v, rows_v, zeros_v, acc_sh,
                       sem0, sem1, sem2, sem3):
        c = lax.axis_index("c")
        s = lax.axis_index("s")
        tables = (t0, t1, t2, t3)
        sems = (sem0, sem1, sem2, sem3)
        zv = jnp.zeros((L,), jnp.float32)

        def zb_body(i, _):
            zeros_v[i, pl.ds(0, L)] = zv
            zeros_v[i, pl.ds(L, L)] = zv
            return ()
        lax.fori_loop(0, ZB, zb_body, (), unroll=False)

        for ch in range(NCHUNK):
            @pl.when(c == ch // 2)
            def _():
                for z in range(ZCH):
                    pltpu.sync_copy(
                        zeros_v, acc_sh.at[pl.ds(s * ZR + z * ZB, ZB)])
                plsc.subcore_barrier()
                tab = tables[ch]

                def sb_body(b, _):
                    pltpu.sync_copy(
                        gidx_hbm.at[s, pl.ds(b * SB, SB)], gidx_v)
                    pltpu.sync_copy(
                        sidx_hbm.at[s, pl.ds(b * SB, SB)], sidx_v)

                    for q in range(NBUF - 1):
                        pltpu.async_copy(
                            tab.at[gidx_v.at[q]], rows_v.at[q], sems[q])

                    def body(jo, _):
                        for p in range(NBUF):
                            j = NBUF * jo + p

                            @pl.when(j + NBUF - 1 < SB)
                            def _():
                                pp = (p + NBUF - 1) % NBUF
                                pltpu.async_copy(
                                    tab.at[gidx_v.at[j + NBUF - 1]],
                                    rows_v.at[pp], sems[pp])
                            pltpu.make_async_copy(
                                tab.at[gidx_v.at[j]], rows_v.at[p],
                                sems[p]).wait()
                            pltpu.sync_copy(
                                rows_v.at[p], acc_sh.at[sidx_v.at[j]],
                                add=True)
                        return ()
                    lax.fori_loop(0, SB // NBUF, body, (), unroll=False)
                    return ()
                lax.fori_loop(0, nsb, sb_body, (), unroll=False)
                plsc.subcore_barrier()
                pltpu.sync_copy(acc_sh.at[pl.ds(s * ZR, ZR)],
                                out_hbm.at[ch, pl.ds(s * ZR, ZR)])
    return scatter_kernel


@functools.lru_cache(maxsize=None)
def make_count_kernel(nbp, nbn):
    @functools.partial(
        pl.kernel,
        out_type=[jax.ShapeDtypeStruct((NS, NACC), jnp.float32),
                  jax.ShapeDtypeStruct((NS, NACC), jnp.float32)],
        mesh=_get_mesh(),
        compiler_params=_CP,
        scratch_types=[
            pltpu.VMEM((nbp, 128), jnp.int32),
            pltpu.VMEM((nbn, 128), jnp.int32),
            pltpu.VMEM((NACC,), jnp.float32),
        ],
    )
    def count_kernel(psidx_hbm, nsidx_hbm, outp_hbm, outn_hbm,
                     pidx_v, nidx_v, cnt_v):
        c = lax.axis_index("c")
        s = lax.axis_index("s")

        def zbody(i, _):
            cnt_v[pl.ds(i * L, L)] = jnp.zeros((L,), jnp.float32)
            return ()
        lax.fori_loop(0, NACC // L, zbody, (), unroll=False)

        ones = jnp.ones((L,), jnp.float32)

        def accumulate(idx_v, nb):
            def body(j, _):
                def inner(k, _):
                    idx = idx_v[j, pl.ds(k * L, L)]
                    plsc.addupdate_scatter(cnt_v, [idx], ones)
                    return ()
                lax.fori_loop(0, 128 // L, inner, (), unroll=False)
                return ()
            lax.fori_loop(0, nb, body, (), unroll=False)

        @pl.when(c == 0)
        def _():
            pltpu.sync_copy(psidx_hbm.at[s], pidx_v)
            accumulate(pidx_v, nbp)
            pltpu.sync_copy(cnt_v, outp_hbm.at[s])

        @pl.when(c == 1)
        def _():
            pltpu.sync_copy(nsidx_hbm.at[s], nidx_v)
            accumulate(nidx_v, nbn)
            pltpu.sync_copy(cnt_v, outn_hbm.at[s])
    return count_kernel


@functools.lru_cache(maxsize=None)
def make_edge_kernel(nb, is_pos):
    @functools.partial(
        pl.kernel,
        out_type=jax.ShapeDtypeStruct((NW, L), jnp.float32),
        mesh=_get_mesh(),
        compiler_params=_CP,
        scratch_types=[
            pltpu.VMEM((nb, 128), jnp.int32),
            pltpu.VMEM((nb, 128), jnp.int32),
            pltpu.VMEM((2, 128, D_FEAT), jnp.float32),
            pltpu.VMEM((2, 128, D_FEAT), jnp.float32),
            pltpu.VMEM((2, 128, 8), jnp.int32),
            pltpu.VMEM((2, 128, 8), jnp.int32),
            pltpu.VMEM((L,), jnp.float32),
            pltpu.SemaphoreType.DMA,
            pltpu.SemaphoreType.DMA,
        ],
    )
    def edge_kernel(iidx_hbm, jidx_hbm, clar_hbm, zn_hbm, out_hbm,
                    iidx_v, jidx_v, ra_v, rb_v, ci_v, cj_v, acc_v,
                    sem0, sem1):
        c = lax.axis_index("c")
        s = lax.axis_index("s")
        wid = c * NS + s
        pltpu.sync_copy(iidx_hbm.at[wid], iidx_v)
        pltpu.sync_copy(jidx_hbm.at[wid], jidx_v)

        lane = lax.iota(jnp.int32, L)
        zero = jnp.zeros((L,), jnp.float32)
        zlane = jnp.zeros((L,), jnp.int32)
        sems = (sem0, sem1)

        def issue(g, b, sem):
            pltpu.async_copy(zn_hbm.at[iidx_v.at[g]], ra_v.at[b], sem)
            pltpu.async_copy(zn_hbm.at[jidx_v.at[g]], rb_v.at[b], sem)
            pltpu.async_copy(clar_hbm.at[iidx_v.at[g]], ci_v.at[b], sem)
            pltpu.async_copy(clar_hbm.at[jidx_v.at[g]], cj_v.at[b], sem)

        def drain(g, b, sem):
            pltpu.make_async_copy(
                zn_hbm.at[iidx_v.at[g]], ra_v.at[b], sem).wait()
            pltpu.make_async_copy(
                zn_hbm.at[jidx_v.at[g]], rb_v.at[b], sem).wait()
            pltpu.make_async_copy(
                clar_hbm.at[iidx_v.at[g]], ci_v.at[b], sem).wait()
            pltpu.make_async_copy(
                clar_hbm.at[jidx_v.at[g]], cj_v.at[b], sem).wait()

        issue(0, 0, sem0)

        def outer(go, acc):
            for b in range(2):
                g = 2 * go + b

                @pl.when(g + 1 < nb)
                def _():
                    issue(g + 1, 1 - b, sems[1 - b])
                drain(g, b, sems[b])

                def group(gg, acc):
                    eidx = gg * L + lane
                    ci = plsc.load_gather(ci_v.at[b], [eidx, zlane])
                    cj = plsc.load_gather(cj_v.at[b], [eidx, zlane])

                    def dloop(d, dot):
                        dv = jnp.full((L,), d, jnp.int32)
                        va = plsc.load_gather(ra_v.at[b], [eidx, dv])
                        vb = plsc.load_gather(rb_v.at[b], [eidx, dv])
                        return dot + va * vb
                    dot = lax.fori_loop(0, D_FEAT, dloop, zero, unroll=8)
                    if is_pos:
                        term = jnp.where(ci != cj, jnp.maximum(dot, 0.0), 0.0)
                    else:
                        term = jnp.where(ci == cj, -jnp.minimum(dot, 0.0), 0.0)
                    return acc + term
                acc = lax.fori_loop(0, 128 // L, group, acc, unroll=False)
            return acc

        acc = lax.fori_loop(0, nb // 2, outer, zero, unroll=False)
        acc_v[...] = acc
        pltpu.sync_copy(acc_v, out_hbm.at[wid])
    return edge_kernel


NPAD = 50048
RB = 2176
GB = NPAD // RB


def _l1_body(x_ref, sp_ref, sn_ref, cp_ref, cn_ref,
             wp_ref, bp_ref, wn_ref, bn_ref, hh_ref):
    x = x_ref[...]
    cp = jnp.maximum(jnp.sum(cp_ref[...], axis=0), 1.0)
    cn = jnp.maximum(jnp.sum(cn_ref[...], axis=0), 1.0)
    op = jnp.concatenate([sp_ref[i] for i in range(NCHUNK)], axis=1)
    on = jnp.concatenate([sn_ref[i] for i in range(NCHUNK)], axis=1)
    op = op / cp[:, None]
    on = on / cn[:, None]

    def head(o, w_ref, b_ref):
        w = w_ref[...]
        a = (jnp.dot(o, w[:D_FEAT], preferred_element_type=jnp.float32)
             + jnp.dot(x, w[D_FEAT:], preferred_element_type=jnp.float32)
             + b_ref[...])
        nrm = jnp.maximum(
            jnp.sqrt(jnp.sum(a * a, axis=1, keepdims=True)), EPS_NORM)
        return jnp.tanh(a / nrm)

    hp = head(op, wp_ref, bp_ref)
    hn = head(on, wn_ref, bn_ref)
    hh_ref[...] = jnp.concatenate([hp, hn], axis=1)


def layer1_tc(x, sp, sn, cp, cn, wp, bp, wn, bn):
    full = lambda shape: pl.BlockSpec(shape, lambda i: tuple(0 for _ in shape))
    return pl.pallas_call(
        _l1_body,
        grid=(GB,),
        in_specs=[
            pl.BlockSpec((RB, D_FEAT), lambda i: (i, 0)),
            pl.BlockSpec((NCHUNK, RB, DC), lambda i: (0, i, 0)),
            pl.BlockSpec((NCHUNK, RB, DC), lambda i: (0, i, 0)),
            pl.BlockSpec((NS, RB), lambda i: (0, i)),
            pl.BlockSpec((NS, RB), lambda i: (0, i)),
            full((2 * D_FEAT, HIDDEN)),
            full((1, HIDDEN)),
            full((2 * D_FEAT, HIDDEN)),
            full((1, HIDDEN)),
        ],
        out_specs=pl.BlockSpec((RB, 2 * HIDDEN), lambda i: (i, 0)),
        out_shape=jax.ShapeDtypeStruct((NPAD, 2 * HIDDEN), jnp.float32),
    )(x, sp, sn, cp, cn, wp, bp.reshape(1, -1), wn, bn.reshape(1, -1))


def _l2_body(hh_ref, b1_ref, b2_ref, cp_ref, cn_ref,
             wp_ref, bp_ref, wn_ref, bn_ref, rw_ref, rb_ref, comm_ref,
             z_ref, zn_ref, clar_ref, rl_ref):
    hh = hh_ref[...]
    hp = hh[:, :HIDDEN]
    hn = hh[:, HIDDEN:]
    cp = jnp.maximum(jnp.sum(cp_ref[...], axis=0) + 1.0, 1.0)[:, None]
    cn = jnp.maximum(jnp.sum(cn_ref[...], axis=0) + 1.0, 1.0)[:, None]
    b1 = jnp.concatenate([b1_ref[i] for i in range(NCHUNK)], axis=1)
    b2 = jnp.concatenate([b2_ref[i] for i in range(NCHUNK)], axis=1)
    o1 = (b1 + hh) / cp
    o2 = (b2 + hh) / cn

    def head(u, w_ref, b_ref):
        a = jnp.dot(u, w_ref[...], preferred_element_type=jnp.float32) \
            + b_ref[...]
        nrm = jnp.maximum(
            jnp.sqrt(jnp.sum(a * a, axis=1, keepdims=True)), EPS_NORM)
        return jnp.tanh(a / nrm)

    hp1 = head(jnp.concatenate(
        [o1[:, :HIDDEN], o2[:, HIDDEN:], hp], axis=1), wp_ref, bp_ref)
    hn1 = head(jnp.concatenate(
        [o1[:, HIDDEN:], o2[:, :HIDDEN], hn], axis=1), wn_ref, bn_ref)
    z = jnp.concatenate([hp1, hn1], axis=1)
    z_ref[...] = z
    znrm = jnp.maximum(
        jnp.sqrt(jnp.sum(z * z, axis=1, keepdims=True)), EPS_COS)
    zn_ref[...] = z / znrm

    preds = jnp.dot(z, rw_ref[...], preferred_element_type=jnp.float32) \
        + rb_ref[...]
    colmask = lax.broadcasted_iota(jnp.int32, preds.shape, 1) < 3
    pm = jnp.where(colmask, preds, -jnp.inf)
    mx = jnp.max(pm, axis=1)
    se = jnp.sum(jnp.where(colmask, jnp.exp(pm - mx[:, None]), 0.0), axis=1)
    clar = jnp.argmax(pm, axis=1).astype(jnp.int32)
    clar_ref[...] = jnp.broadcast_to(clar[:, None], (RB, 8))
    pick_mask = (lax.broadcasted_iota(jnp.int32, preds.shape, 1)
                 == comm_ref[...])
    pick = jnp.sum(jnp.where(pick_mask, preds, 0.0), axis=1)
    row = (pl.program_id(0) * RB
           + lax.broadcasted_iota(jnp.int32, (RB, 1), 0))
    val = jnp.sum(jnp.where(
        row < N_NODES, (mx + jnp.log(se) - pick)[:, None], 0.0))

    @pl.when(pl.program_id(0) == 0)
    def _():
        rl_ref[...] = jnp.zeros((1, 1), jnp.float32)
    rl_ref[...] += jnp.full((1, 1), val, jnp.float32)


def layer2_tc(hh, b1, b2, cp, cn, wp, bp, wn, bn, rw, rbias, comm):
    full = lambda shape: pl.BlockSpec(shape, lambda i: tuple(0 for _ in shape))
    return pl.pallas_call(
        _l2_body,
        grid=(GB,),
        in_specs=[
            pl.BlockSpec((RB, 2 * HIDDEN), lambda i: (i, 0)),
            pl.BlockSpec((NCHUNK, RB, DC), lambda i: (0, i, 0)),
            pl.BlockSpec((NCHUNK, RB, DC), lambda i: (0, i, 0)),
            pl.BlockSpec((NS, RB), lambda i: (0, i)),
            pl.BlockSpec((NS, RB), lambda i: (0, i)),
            full((3 * HIDDEN, HIDDEN)),
            full((1, HIDDEN)),
            full((3 * HIDDEN, HIDDEN)),
            full((1, HIDDEN)),
            full((2 * HIDDEN, 128)),
            full((1, 128)),
            pl.BlockSpec((RB, 1), lambda i: (i, 0)),
        ],
        out_specs=[
            pl.BlockSpec((RB, 2 * HIDDEN), lambda i: (i, 0)),
            pl.BlockSpec((RB, 2 * HIDDEN), lambda i: (i, 0)),
            pl.BlockSpec((RB, 8), lambda i: (i, 0)),
            pl.BlockSpec((1, 1), lambda i: (0, 0)),
        ],
        out_shape=[
            jax.ShapeDtypeStruct((NPAD, 2 * HIDDEN), jnp.float32),
            jax.ShapeDtypeStruct((NPAD, 2 * HIDDEN), jnp.float32),
            jax.ShapeDtypeStruct((NPAD, 8), jnp.int32),
            jax.ShapeDtypeStruct((1, 1), jnp.float32),
        ],
    )(hh, b1, b2, cp, cn, wp, bp.reshape(1, -1), wn, bn.reshape(1, -1),
      rw, rbias.reshape(1, -1), comm)


def finalize_tc(rl, pparts, nparts, e_pos, e_neg):
    def body(rl_ref, pp_ref, np_ref, out_ref):
        reg = jnp.sum(rl_ref[...]) / N_NODES
        sim1 = jnp.sum(pp_ref[...]) / e_pos
        sim2 = jnp.sum(np_ref[...]) / e_neg
        out_ref[...] = jnp.full(
            (1, 1), LAMB * reg + (1.0 - LAMB) * (sim1 + sim2), jnp.float32)
    return pl.pallas_call(
        body,
        out_shape=jax.ShapeDtypeStruct((1, 1), jnp.float32),
    )(rl, pparts, nparts)


def _pad_to(a, n, fill):
    return jnp.concatenate(
        [a, jnp.full((n - a.shape[0],), fill, a.dtype)])


def _chunk_table(t):
    tc = jnp.moveaxis(t.reshape(t.shape[0], NCHUNK, DC), 1, 0)
    return [tc[i] for i in range(NCHUNK)]


def kernel(positive_edges, negative_edges, comm, X,
           W_pos_base, b_pos_base, W_neg_base, b_neg_base,
           W_pos_deep, b_pos_deep, W_neg_deep, b_neg_deep,
           regression_weights, regression_bias):
    e_pos = positive_edges.shape[1]
    e_neg = negative_edges.shape[1]

    def prep_scatter(edges):
        e = edges.shape[1]
        gran = NS * 128 * SB
        epad = -(-e // gran) * gran
        gidx = _pad_to(edges[1], epad, 0)
        sidx = jnp.where(edges[0] == edges[1], DUMMY, edges[0])
        sidx = _pad_to(sidx, epad, DUMMY)
        nb = epad // (NS * 128)
        return (gidx.reshape(NS, nb, 128), sidx.reshape(NS, nb, 128),
                nb // SB)

    pg, ps, nsbp = prep_scatter(positive_edges)
    ng, nsx, nsbn = prep_scatter(negative_edges)

    cntp, cntn = make_count_kernel(nsbp * SB, nsbn * SB)(ps, nsx)

    xch = _chunk_table(X)
    sp = make_scatter_kernel(nsbp)(pg, ps, *xch)
    sn = make_scatter_kernel(nsbn)(ng, nsx, *xch)

    xp = jnp.concatenate(
        [X, jnp.zeros((NPAD - N_NODES, D_FEAT), jnp.float32)])
    hh = layer1_tc(xp, sp, sn, cntp, cntn,
                   W_pos_base, b_pos_base, W_neg_base, b_neg_base)

    hch = _chunk_table(hh)
    b1 = make_scatter_kernel(nsbp)(pg, ps, *hch)
    b2 = make_scatter_kernel(nsbn)(ng, nsx, *hch)

    rwpad = jnp.zeros((2 * HIDDEN, 128), jnp.float32).at[:, :3].set(
        regression_weights)
    rbpad = jnp.zeros((128,), jnp.float32).at[:3].set(regression_bias)
    commp = _pad_to(comm.astype(jnp.int32), NPAD, 0).reshape(-1, 1)
    z, zn, clar2d, rl = layer2_tc(
        hh, b1, b2, cntp, cntn,
        W_pos_deep, b_pos_deep, W_neg_deep, b_neg_deep,
        rwpad, rbpad, commp)

    def prep_edge(edges):
        e = edges.shape[1]
        gran = NW * 128 * 2
        epad = -(-e // gran) * gran
        i = _pad_to(edges[0], epad, 0)
        j = _pad_to(edges[1], epad, 0)
        nb = epad // (NW * 128)
        return (i.reshape(NW, nb, 128), j.reshape(NW, nb, 128), nb)

    pi, pj, nbp2 = prep_edge(positive_edges)
    ni, nj, nbn2 = prep_edge(negative_edges)
    pparts = make_edge_kernel(nbp2, True)(pi, pj, clar2d, zn)
    nparts = make_edge_kernel(nbn2, False)(ni, nj, clar2d, zn)

    clar1d = clar2d[:N_NODES, 0]

    loss = finalize_tc(rl, pparts, nparts, e_pos, e_neg)
    return loss[0, 0], z[:N_NODES], clar1d


# speedup vs baseline: 5.0800x; 1.0005x over previous
import functools
import jax
import jax.numpy as jnp
from jax import lax
from jax.experimental import pallas as pl
from jax.experimental.pallas import tpu as pltpu
from jax.experimental.pallas import tpu_sc as plsc

N_NODES = 50000
D_FEAT = 128
HIDDEN = 64
LAMB = 0.8
EPS_NORM = 1e-12
EPS_COS = 1e-8

NC, NS, L = 2, 16, 16
NW = NC * NS
NACC = 50048
DUMMY = N_NODES
ZCH = 46
ZR = NACC // NS
ZB = ZR // ZCH
NBUF = 4
NCHUNK = 4
DC = 32
SB = 8

@functools.cache
def _get_mesh():
    return plsc.VectorSubcoreMesh(core_axis_name="c", subcore_axis_name="s")
_CP = pltpu.CompilerParams(use_tc_tiling_on_sc=False,
                           needs_layout_passes=False)


@functools.lru_cache(maxsize=None)
def make_scatter_kernel(nsb):
    @functools.partial(
        pl.kernel,
        out_type=jax.ShapeDtypeStruct((NCHUNK, NACC, DC), jnp.float32),
        mesh=_get_mesh(),
        compiler_params=_CP,
        scratch_types=[
            pltpu.VMEM((SB, 128), jnp.int32),
            pltpu.VMEM((SB, 128), jnp.int32),
            pltpu.VMEM((NBUF, 128, DC), jnp.float32),
            pltpu.VMEM((ZB, DC), jnp.float32),
            pltpu.VMEM_SHARED((NACC, DC), jnp.float32),
            pltpu.SemaphoreType.DMA,
            pltpu.SemaphoreType.DMA,
            pltpu.SemaphoreType.DMA,
            pltpu.SemaphoreType.DMA,
        ],
    )
    def scatter_kernel(gidx_hbm, sidx_hbm, t0, t1, t2, t3, out_hbm,
                       gidx_v, sidx_v, rows_v, zeros_v, acc_sh,
                       sem0, sem1, sem2, sem3):
        c = lax.axis_index("c")
        s = lax.axis_index("s")
        tables = (t0, t1, t2, t3)
        sems = (sem0, sem1, sem2, sem3)
        zv = jnp.zeros((L,), jnp.float32)

        def zb_body(i, _):
            zeros_v[i, pl.ds(0, L)] = zv
            zeros_v[i, pl.ds(L, L)] = zv
            return ()
        lax.fori_loop(0, ZB, zb_body, (), unroll=False)

        for ch in range(NCHUNK):
            @pl.when(c == ch // 2)
            def _():
                for z in range(ZCH):
                    pltpu.sync_copy(
                        zeros_v, acc_sh.at[pl.ds(s * ZR + z * ZB, ZB)])
                plsc.subcore_barrier()
                tab = tables[ch]

                def sb_body(b, _):
                    pltpu.sync_copy(
                        gidx_hbm.at[s, pl.ds(b * SB, SB)], gidx_v)
                    pltpu.sync_copy(
                        sidx_hbm.at[s, pl.ds(b * SB, SB)], sidx_v)

                    for q in range(NBUF - 1):
                        pltpu.async_copy(
                            tab.at[gidx_v.at[q]], rows_v.at[q], sems[q])

                    def body(jo, _):
                        for p in range(NBUF):
                            j = NBUF * jo + p

                            @pl.when(j + NBUF - 1 < SB)
                            def _():
                                pp = (p + NBUF - 1) % NBUF
                                pltpu.async_copy(
                                    tab.at[gidx_v.at[j + NBUF - 1]],
                                    rows_v.at[pp], sems[pp])
                            pltpu.make_async_copy(
                                tab.at[gidx_v.at[j]], rows_v.at[p],
                                sems[p]).wait()
                            pltpu.sync_copy(
                                rows_v.at[p], acc_sh.at[sidx_v.at[j]],
                                add=True)
                        return ()
                    lax.fori_loop(0, SB // NBUF, body, (), unroll=False)
                    return ()
                lax.fori_loop(0, nsb, sb_body, (), unroll=False)
                plsc.subcore_barrier()
                pltpu.sync_copy(acc_sh.at[pl.ds(s * ZR, ZR)],
                                out_hbm.at[ch, pl.ds(s * ZR, ZR)])
    return scatter_kernel


@functools.lru_cache(maxsize=None)
def make_count_kernel(nbp, nbn):
    @functools.partial(
        pl.kernel,
        out_type=[jax.ShapeDtypeStruct((NS, NACC), jnp.float32),
                  jax.ShapeDtypeStruct((NS, NACC), jnp.float32)],
        mesh=_get_mesh(),
        compiler_params=_CP,
        scratch_types=[
            pltpu.VMEM((nbp, 128), jnp.int32),
            pltpu.VMEM((nbn, 128), jnp.int32),
            pltpu.VMEM((NACC,), jnp.float32),
        ],
    )
    def count_kernel(psidx_hbm, nsidx_hbm, outp_hbm, outn_hbm,
                     pidx_v, nidx_v, cnt_v):
        c = lax.axis_index("c")
        s = lax.axis_index("s")

        def zbody(i, _):
            cnt_v[pl.ds(i * L, L)] = jnp.zeros((L,), jnp.float32)
            return ()
        lax.fori_loop(0, NACC // L, zbody, (), unroll=False)

        ones = jnp.ones((L,), jnp.float32)

        def accumulate(idx_v, nb):
            def body(j, _):
                def inner(k, _):
                    idx = idx_v[j, pl.ds(k * L, L)]
                    plsc.addupdate_scatter(cnt_v, [idx], ones)
                    return ()
                lax.fori_loop(0, 128 // L, inner, (), unroll=False)
                return ()
            lax.fori_loop(0, nb, body, (), unroll=False)

        @pl.when(c == 0)
        def _():
            pltpu.sync_copy(psidx_hbm.at[s], pidx_v)
            accumulate(pidx_v, nbp)
            pltpu.sync_copy(cnt_v, outp_hbm.at[s])

        @pl.when(c == 1)
        def _():
            pltpu.sync_copy(nsidx_hbm.at[s], nidx_v)
            accumulate(nidx_v, nbn)
            pltpu.sync_copy(cnt_v, outn_hbm.at[s])
    return count_kernel


@functools.lru_cache(maxsize=None)
def make_edge_kernel(nb, is_pos):
    @functools.partial(
        pl.kernel,
        out_type=jax.ShapeDtypeStruct((NW, L), jnp.float32),
        mesh=_get_mesh(),
        compiler_params=_CP,
        scratch_types=[
            pltpu.VMEM((nb, 128), jnp.int32),
            pltpu.VMEM((nb, 128), jnp.int32),
            pltpu.VMEM((2, 128, D_FEAT), jnp.float32),
            pltpu.VMEM((2, 128, D_FEAT), jnp.float32),
            pltpu.VMEM((2, 128, 8), jnp.int32),
            pltpu.VMEM((2, 128, 8), jnp.int32),
            pltpu.VMEM((L,), jnp.float32),
            pltpu.SemaphoreType.DMA,
            pltpu.SemaphoreType.DMA,
        ],
    )
    def edge_kernel(iidx_hbm, jidx_hbm, clar_hbm, zn_hbm, out_hbm,
                    iidx_v, jidx_v, ra_v, rb_v, ci_v, cj_v, acc_v,
                    sem0, sem1):
        c = lax.axis_index("c")
        s = lax.axis_index("s")
        wid = c * NS + s
        pltpu.sync_copy(iidx_hbm.at[wid], iidx_v)
        pltpu.sync_copy(jidx_hbm.at[wid], jidx_v)

        lane = lax.iota(jnp.int32, L)
        zero = jnp.zeros((L,), jnp.float32)
        zlane = jnp.zeros((L,), jnp.int32)
        sems = (sem0, sem1)

        def issue(g, b, sem):
            pltpu.async_copy(zn_hbm.at[iidx_v.at[g]], ra_v.at[b], sem)
            pltpu.async_copy(zn_hbm.at[jidx_v.at[g]], rb_v.at[b], sem)
            pltpu.async_copy(clar_hbm.at[iidx_v.at[g]], ci_v.at[b], sem)
            pltpu.async_copy(clar_hbm.at[jidx_v.at[g]], cj_v.at[b], sem)

        def drain(g, b, sem):
            pltpu.make_async_copy(
                zn_hbm.at[iidx_v.at[g]], ra_v.at[b], sem).wait()
            pltpu.make_async_copy(
                zn_hbm.at[jidx_v.at[g]], rb_v.at[b], sem).wait()
            pltpu.make_async_copy(
                clar_hbm.at[iidx_v.at[g]], ci_v.at[b], sem).wait()
            pltpu.make_async_copy(
                clar_hbm.at[jidx_v.at[g]], cj_v.at[b], sem).wait()

        issue(0, 0, sem0)

        def outer(go, acc):
            for b in range(2):
                g = 2 * go + b

                @pl.when(g + 1 < nb)
                def _():
                    issue(g + 1, 1 - b, sems[1 - b])
                drain(g, b, sems[b])

                def group(gg, acc):
                    eidx = gg * L + lane
                    ci = plsc.load_gather(ci_v.at[b], [eidx, zlane])
                    cj = plsc.load_gather(cj_v.at[b], [eidx, zlane])

                    def dloop(d, dot):
                        dv = jnp.full((L,), d, jnp.int32)
                        va = plsc.load_gather(ra_v.at[b], [eidx, dv])
                        vb = plsc.load_gather(rb_v.at[b], [eidx, dv])
                        return dot + va * vb
                    dot = lax.fori_loop(0, D_FEAT, dloop, zero, unroll=8)
                    if is_pos:
                        term = jnp.where(ci != cj, jnp.maximum(dot, 0.0), 0.0)
                    else:
                        term = jnp.where(ci == cj, -jnp.minimum(dot, 0.0), 0.0)
                    return acc + term
                acc = lax.fori_loop(0, 128 // L, group, acc, unroll=False)
            return acc

        acc = lax.fori_loop(0, nb // 2, outer, zero, unroll=False)
        acc_v[...] = acc
        pltpu.sync_copy(acc_v, out_hbm.at[wid])
    return edge_kernel


NPAD = 50048
RB = 2176
GB = NPAD // RB


def _l1_body(x_ref, sp_ref, sn_ref, cp_ref, cn_ref,
             wp_ref, bp_ref, wn_ref, bn_ref, hh_ref):
    x = x_ref[...]
    cp = jnp.maximum(jnp.sum(cp_ref[...], axis=0), 1.0)
    cn = jnp.maximum(jnp.sum(cn_ref[...], axis=0), 1.0)
    op = jnp.concatenate([sp_ref[i] for i in range(NCHUNK)], axis=1)
    on = jnp.concatenate([sn_ref[i] for i in range(NCHUNK)], axis=1)
    op = op / cp[:, None]
    on = on / cn[:, None]

    def head(o, w_ref, b_ref):
        w = w_ref[...]
        a = (jnp.dot(o, w[:D_FEAT], preferred_element_type=jnp.float32)
             + jnp.dot(x, w[D_FEAT:], preferred_element_type=jnp.float32)
             + b_ref[...])
        nrm = jnp.maximum(
            jnp.sqrt(jnp.sum(a * a, axis=1, keepdims=True)), EPS_NORM)
        return jnp.tanh(a / nrm)

    hp = head(op, wp_ref, bp_ref)
    hn = head(on, wn_ref, bn_ref)
    hh_ref[...] = jnp.concatenate([hp, hn], axis=1)


def layer1_tc(x, sp, sn, cp, cn, wp, bp, wn, bn):
    full = lambda shape: pl.BlockSpec(shape, lambda i: tuple(0 for _ in shape))
    return pl.pallas_call(
        _l1_body,
        grid=(GB,),
        in_specs=[
            pl.BlockSpec((RB, D_FEAT), lambda i: (i, 0)),
            pl.BlockSpec((NCHUNK, RB, DC), lambda i: (0, i, 0)),
            pl.BlockSpec((NCHUNK, RB, DC), lambda i: (0, i, 0)),
            pl.BlockSpec((NS, RB), lambda i: (0, i)),
            pl.BlockSpec((NS, RB), lambda i: (0, i)),
            full((2 * D_FEAT, HIDDEN)),
            full((1, HIDDEN)),
            full((2 * D_FEAT, HIDDEN)),
            full((1, HIDDEN)),
        ],
        out_specs=pl.BlockSpec((RB, 2 * HIDDEN), lambda i: (i, 0)),
        out_shape=jax.ShapeDtypeStruct((NPAD, 2 * HIDDEN), jnp.float32),
    )(x, sp, sn, cp, cn, wp, bp.reshape(1, -1), wn, bn.reshape(1, -1))


def _l2_body(hh_ref, b1_ref, b2_ref, cp_ref, cn_ref,
             wp_ref, bp_ref, wn_ref, bn_ref, rw_ref, rb_ref, comm_ref,
             z_ref, zn_ref, clar_ref, rl_ref):
    hh = hh_ref[...]
    hp = hh[:, :HIDDEN]
    hn = hh[:, HIDDEN:]
    cp = jnp.maximum(jnp.sum(cp_ref[...], axis=0) + 1.0, 1.0)[:, None]
    cn = jnp.maximum(jnp.sum(cn_ref[...], axis=0) + 1.0, 1.0)[:, None]
    b1 = jnp.concatenate([b1_ref[i] for i in range(NCHUNK)], axis=1)
    b2 = jnp.concatenate([b2_ref[i] for i in range(NCHUNK)], axis=1)
    o1 = (b1 + hh) / cp
    o2 = (b2 + hh) / cn

    def head(u, w_ref, b_ref):
        a = jnp.dot(u, w_ref[...], preferred_element_type=jnp.float32) \
            + b_ref[...]
        nrm = jnp.maximum(
            jnp.sqrt(jnp.sum(a * a, axis=1, keepdims=True)), EPS_NORM)
        return jnp.tanh(a / nrm)

    hp1 = head(jnp.concatenate(
        [o1[:, :HIDDEN], o2[:, HIDDEN:], hp], axis=1), wp_ref, bp_ref)
    hn1 = head(jnp.concatenate(
        [o1[:, HIDDEN:], o2[:, :HIDDEN], hn], axis=1), wn_ref, bn_ref)
    z = jnp.concatenate([hp1, hn1], axis=1)
    z_ref[...] = z
    znrm = jnp.maximum(
        jnp.sqrt(jnp.sum(z * z, axis=1, keepdims=True)), EPS_COS)

    preds = jnp.dot(z, rw_ref[...], preferred_element_type=jnp.float32) \
        + rb_ref[...]
    colmask = lax.broadcasted_iota(jnp.int32, preds.shape, 1) < 3
    pm = jnp.where(colmask, preds, -jnp.inf)
    mx = jnp.max(pm, axis=1)
    se = jnp.sum(jnp.where(colmask, jnp.exp(pm - mx[:, None]), 0.0), axis=1)
    clar = jnp.argmax(pm, axis=1).astype(jnp.int32)
    clar_ref[...] = jnp.broadcast_to(clar[:, None], (RB, 8))
    zn_ref[...] = z / znrm
    pick_mask = (lax.broadcasted_iota(jnp.int32, preds.shape, 1)
                 == comm_ref[...])
    pick = jnp.sum(jnp.where(pick_mask, preds, 0.0), axis=1)
    row = (pl.program_id(0) * RB
           + lax.broadcasted_iota(jnp.int32, (RB, 1), 0))
    val = jnp.sum(jnp.where(
        row < N_NODES, (mx + jnp.log(se) - pick)[:, None], 0.0))

    @pl.when(pl.program_id(0) == 0)
    def _():
        rl_ref[...] = jnp.zeros((1, 1), jnp.float32)
    rl_ref[...] += jnp.full((1, 1), val, jnp.float32)


def layer2_tc(hh, b1, b2, cp, cn, wp, bp, wn, bn, rw, rbias, comm):
    full = lambda shape: pl.BlockSpec(shape, lambda i: tuple(0 for _ in shape))
    return pl.pallas_call(
        _l2_body,
        grid=(GB,),
        in_specs=[
            pl.BlockSpec((RB, 2 * HIDDEN), lambda i: (i, 0)),
            pl.BlockSpec((NCHUNK, RB, DC), lambda i: (0, i, 0)),
            pl.BlockSpec((NCHUNK, RB, DC), lambda i: (0, i, 0)),
            pl.BlockSpec((NS, RB), lambda i: (0, i)),
            pl.BlockSpec((NS, RB), lambda i: (0, i)),
            full((3 * HIDDEN, HIDDEN)),
            full((1, HIDDEN)),
            full((3 * HIDDEN, HIDDEN)),
            full((1, HIDDEN)),
            full((2 * HIDDEN, 128)),
            full((1, 128)),
            pl.BlockSpec((RB, 1), lambda i: (i, 0)),
        ],
        out_specs=[
            pl.BlockSpec((RB, 2 * HIDDEN), lambda i: (i, 0)),
            pl.BlockSpec((RB, 2 * HIDDEN), lambda i: (i, 0)),
            pl.BlockSpec((RB, 8), lambda i: (i, 0)),
            pl.BlockSpec((1, 1), lambda i: (0, 0)),
        ],
        out_shape=[
            jax.ShapeDtypeStruct((NPAD, 2 * HIDDEN), jnp.float32),
            jax.ShapeDtypeStruct((NPAD, 2 * HIDDEN), jnp.float32),
            jax.ShapeDtypeStruct((NPAD, 8), jnp.int32),
            jax.ShapeDtypeStruct((1, 1), jnp.float32),
        ],
    )(hh, b1, b2, cp, cn, wp, bp.reshape(1, -1), wn, bn.reshape(1, -1),
      rw, rbias.reshape(1, -1), comm)


def finalize_tc(rl, pparts, nparts, e_pos, e_neg):
    def body(rl_ref, pp_ref, np_ref, out_ref):
        reg = jnp.sum(rl_ref[...]) / N_NODES
        sim1 = jnp.sum(pp_ref[...]) / e_pos
        sim2 = jnp.sum(np_ref[...]) / e_neg
        out_ref[...] = jnp.full(
            (1, 1), LAMB * reg + (1.0 - LAMB) * (sim1 + sim2), jnp.float32)
    return pl.pallas_call(
        body,
        out_shape=jax.ShapeDtypeStruct((1, 1), jnp.float32),
    )(rl, pparts, nparts)


def _pad_to(a, n, fill):
    return jnp.concatenate(
        [a, jnp.full((n - a.shape[0],), fill, a.dtype)])


def _chunk_table(t):
    tc = jnp.moveaxis(t.reshape(t.shape[0], NCHUNK, DC), 1, 0)
    return [tc[i] for i in range(NCHUNK)]


def kernel(positive_edges, negative_edges, comm, X,
           W_pos_base, b_pos_base, W_neg_base, b_neg_base,
           W_pos_deep, b_pos_deep, W_neg_deep, b_neg_deep,
           regression_weights, regression_bias):
    e_pos = positive_edges.shape[1]
    e_neg = negative_edges.shape[1]

    def prep_scatter(edges):
        e = edges.shape[1]
        gran = NS * 128 * SB
        epad = -(-e // gran) * gran
        gidx = _pad_to(edges[1], epad, 0)
        sidx = jnp.where(edges[0] == edges[1], DUMMY, edges[0])
        sidx = _pad_to(sidx, epad, DUMMY)
        nb = epad // (NS * 128)
        return (gidx.reshape(NS, nb, 128), sidx.reshape(NS, nb, 128),
                nb // SB)

    pg, ps, nsbp = prep_scatter(positive_edges)
    ng, nsx, nsbn = prep_scatter(negative_edges)

    cntp, cntn = make_count_kernel(nsbp * SB, nsbn * SB)(ps, nsx)

    xch = _chunk_table(X)
    sp = make_scatter_kernel(nsbp)(pg, ps, *xch)
    sn = make_scatter_kernel(nsbn)(ng, nsx, *xch)

    xp = jnp.concatenate(
        [X, jnp.zeros((NPAD - N_NODES, D_FEAT), jnp.float32)])
    hh = layer1_tc(xp, sp, sn, cntp, cntn,
                   W_pos_base, b_pos_base, W_neg_base, b_neg_base)

    hch = _chunk_table(hh)
    b1 = make_scatter_kernel(nsbp)(pg, ps, *hch)
    b2 = make_scatter_kernel(nsbn)(ng, nsx, *hch)

    rwpad = jnp.zeros((2 * HIDDEN, 128), jnp.float32).at[:, :3].set(
        regression_weights)
    rbpad = jnp.zeros((128,), jnp.float32).at[:3].set(regression_bias)
    commp = _pad_to(comm.astype(jnp.int32), NPAD, 0).reshape(-1, 1)
    z, zn, clar2d, rl = layer2_tc(
        hh, b1, b2, cntp, cntn,
        W_pos_deep, b_pos_deep, W_neg_deep, b_neg_deep,
        rwpad, rbpad, commp)

    def prep_edge(edges):
        e = edges.shape[1]
        gran = NW * 128 * 2
        epad = -(-e // gran) * gran
        i = _pad_to(edges[0], epad, 0)
        j = _pad_to(edges[1], epad, 0)
        nb = epad // (NW * 128)
        return (i.reshape(NW, nb, 128), j.reshape(NW, nb, 128), nb)

    pi, pj, nbp2 = prep_edge(positive_edges)
    ni, nj, nbn2 = prep_edge(negative_edges)
    pparts = make_edge_kernel(nbp2, True)(pi, pj, clar2d, zn)
    nparts = make_edge_kernel(nbn2, False)(ni, nj, clar2d, zn)

    clar1d = clar2d[:N_NODES, 0]

    loss = finalize_tc(rl, pparts, nparts, e_pos, e_neg)
    return loss[0, 0], z[:N_NODES], clar1d
